# baseline (speedup 1.0000x reference)
import functools

import jax
import jax.numpy as jnp
from jax import lax
from jax.experimental import pallas as pl
from jax.experimental.pallas import tpu as pltpu
from jax.experimental.pallas import tpu_sc as plsc

N_T = 10000
N_S = 160000
FEAT = 64
ALPHA = 0.5
ROWS = 2000

_USE_SC_PROP = True


def _dense_body(nx, stats, *refs):
    xs = refs[:nx]
    ws = refs[nx:2 * nx]
    b_ref = refs[2 * nx]
    y_ref = refs[2 * nx + 1]
    y = jnp.dot(xs[0][...], ws[0][...], preferred_element_type=jnp.float32)
    for i in range(1, nx):
        y = y + jnp.dot(xs[i][...], ws[i][...],
                        preferred_element_type=jnp.float32)
    y = y + b_ref[...]
    y_ref[...] = y
    if stats:
        st_ref = refs[2 * nx + 2]

        @pl.when(pl.program_id(0) == 0)
        def _():
            st_ref[...] = jnp.zeros_like(st_ref)

        s1 = jnp.sum(y, axis=0, keepdims=True)
        s2 = jnp.sum(y * y, axis=0, keepdims=True)
        st_ref[...] += jnp.concatenate([s1, s2], axis=0)


def _dense(x_list, w_list, b, stats):
    n = x_list[0].shape[0]
    nx = len(x_list)
    cout = w_list[0].shape[1]
    grid = n // ROWS
    in_specs = (
        [pl.BlockSpec((ROWS, x.shape[1]), lambda i: (i, 0)) for x in x_list]
        + [pl.BlockSpec(w.shape, lambda i: (0, 0)) for w in w_list]
        + [pl.BlockSpec((1, cout), lambda i: (0, 0))]
    )
    out_shape = [jax.ShapeDtypeStruct((n, cout), jnp.float32)]
    out_specs = [pl.BlockSpec((ROWS, cout), lambda i: (i, 0))]
    if stats:
        out_shape.append(jax.ShapeDtypeStruct((2, cout), jnp.float32))
        out_specs.append(pl.BlockSpec((2, cout), lambda i: (0, 0)))
    out = pl.pallas_call(
        functools.partial(_dense_body, nx, stats),
        grid=(grid,),
        in_specs=in_specs,
        out_specs=out_specs if stats else out_specs[0],
        out_shape=out_shape if stats else out_shape[0],
    )(*x_list, *w_list, b.reshape(1, cout))
    return out if stats else (out, None)


def _bn_body(n, mix, colsum, *refs):
    y_ref, st_ref, g_ref, bt_ref = refs[:4]
    idx = 4
    if mix:
        x0_ref = refs[idx]
        idx += 1
    o_ref = refs[idx]
    idx += 1
    st = st_ref[...]
    m = st[0:1, :] / n
    v = st[1:2, :] / n - m * m
    scale = g_ref[...] * lax.rsqrt(v + 1e-5)
    o = (y_ref[...] - m) * scale + bt_ref[...]
    o = jnp.maximum(o, 0.0)
    if mix:
        o = (1.0 - ALPHA) * o + ALPHA * x0_ref[...]
    o_ref[...] = o
    if colsum:
        cs_ref = refs[idx]

        @pl.when(pl.program_id(0) == 0)
        def _():
            cs_ref[...] = jnp.zeros_like(cs_ref)

        s1 = jnp.sum(o, axis=0, keepdims=True)
        cs_ref[...] += jnp.concatenate([s1, s1], axis=0)


def _bn_apply(y, st, g, bt, x0=None, colsum=False):
    n, c = y.shape
    grid = n // ROWS
    mix = x0 is not None
    in_specs = [
        pl.BlockSpec((ROWS, c), lambda i: (i, 0)),
        pl.BlockSpec((2, c), lambda i: (0, 0)),
        pl.BlockSpec((1, c), lambda i: (0, 0)),
        pl.BlockSpec((1, c), lambda i: (0, 0)),
    ]
    args = [y, st, g.reshape(1, c), bt.reshape(1, c)]
    if mix:
        in_specs.append(pl.BlockSpec((ROWS, c), lambda i: (i, 0)))
        args.append(x0)
    out_shape = [jax.ShapeDtypeStruct((n, c), jnp.float32)]
    out_specs = [pl.BlockSpec((ROWS, c), lambda i: (i, 0))]
    if colsum:
        out_shape.append(jax.ShapeDtypeStruct((2, c), jnp.float32))
        out_specs.append(pl.BlockSpec((2, c), lambda i: (0, 0)))
    out = pl.pallas_call(
        functools.partial(_bn_body, float(n), mix, colsum),
        grid=(grid,),
        in_specs=in_specs,
        out_specs=out_specs if colsum else out_specs[0],
        out_shape=out_shape if colsum else out_shape[0],
    )(*args)
    return out if colsum else (out, None)


def _scale_rows_body(a_ref, d_ref, o_ref):
    o_ref[...] = a_ref[...] / jnp.maximum(d_ref[...], 1.0)


def _scale_rows(a, d):
    n, c = a.shape
    return pl.pallas_call(
        _scale_rows_body,
        grid=(n // ROWS,),
        in_specs=[pl.BlockSpec((ROWS, c), lambda i: (i, 0)),
                  pl.BlockSpec((ROWS, c), lambda i: (i, 0))],
        out_specs=pl.BlockSpec((ROWS, c), lambda i: (i, 0)),
        out_shape=jax.ShapeDtypeStruct((n, c), jnp.float32),
    )(a, d)


def _final_body(ss_ref, st_ref, w_ref, b_ref, o_ref):
    pooled = jnp.concatenate(
        [ss_ref[0:1, :] / N_S, st_ref[0:1, :] / N_T], axis=1)
    pooled = jnp.broadcast_to(pooled, (8, 2 * FEAT))
    o_ref[...] = jnp.dot(pooled, w_ref[...],
                         preferred_element_type=jnp.float32) + b_ref[...]


def _final(sum_s, sum_t, w16, b16):
    return pl.pallas_call(
        _final_body,
        in_specs=[pl.BlockSpec((2, FEAT), lambda: (0, 0)),
                  pl.BlockSpec((2, FEAT), lambda: (0, 0)),
                  pl.BlockSpec((2 * FEAT, 16), lambda: (0, 0)),
                  pl.BlockSpec((1, 16), lambda: (0, 0))],
        out_specs=pl.BlockSpec((8, 16), lambda: (0, 0)),
        out_shape=jax.ShapeDtypeStruct((8, 16), jnp.float32),
    )(sum_s, sum_t, w16, b16)


def _prop_jnp(x, src, dst, ew):
    msg = ew[:, None] * x[src]
    return jnp.zeros_like(x).at[dst].add(msg)


_NC = 2
_NS = 16
_CHUNK = 2048
_BATCH = 128


@functools.lru_cache(maxsize=None)
def _make_bucketize(e_pad, nb, blk):
    nw = _NC * _NS
    tw = e_pad // nw
    c_sz = 2048 if tw % 2048 == 0 else 1024
    assert tw % c_sz == 0 and tw % 128 == 0
    nchunks = tw // c_sz
    nsg = c_sz // 64
    capr = tw + 128
    out_sz = nw * nb * capr
    pcap = 192
    mesh = plsc.VectorSubcoreMesh(core_axis_name="c", subcore_axis_name="s",
                                  num_cores=_NC, num_subcores=_NS)

    @functools.partial(
        pl.kernel,
        out_type=(jax.ShapeDtypeStruct((out_sz,), jnp.int32),
                  jax.ShapeDtypeStruct((out_sz,), jnp.int32),
                  jax.ShapeDtypeStruct((out_sz,), jnp.float32),
                  jax.ShapeDtypeStruct((nw * 16,), jnp.int32)),
        mesh=mesh,
        compiler_params=pltpu.CompilerParams(use_tc_tiling_on_sc=False,
                                             needs_layout_passes=False),
        scratch_types=dict(
            srcb=pltpu.VMEM((c_sz,), jnp.int32),
            dstb=pltpu.VMEM((c_sz,), jnp.int32),
            ewb=pltpu.VMEM((c_sz,), jnp.float32),
            ps=pltpu.VMEM((nb, pcap), jnp.int32),
            pd=pltpu.VMEM((nb, pcap), jnp.int32),
            pw=pltpu.VMEM((nb, pcap), jnp.float32),
            cbuf=pltpu.VMEM((16,), jnp.int32),
        ),
    )
    def bk(src_hbm, dst_hbm, ew_hbm, bsrc, bdst, bew, cnts,
           srcb, dstb, ewb, ps, pd, pw, cbuf):
        cid = lax.axis_index("c")
        sid = lax.axis_index("s")
        w = sid * _NC + cid
        toff = w * tw

        def flush(b, nfl):
            base = pl.multiple_of((w * nb + b) * capr + nfl * 128, 128)
            pltpu.sync_copy(ps.at[b, pl.ds(0, 128)],
                            bsrc.at[pl.ds(base, 128)])
            pltpu.sync_copy(pd.at[b, pl.ds(0, 128)],
                            bdst.at[pl.ds(base, 128)])
            pltpu.sync_copy(pw.at[b, pl.ds(0, 128)],
                            bew.at[pl.ds(base, 128)])

        def shift(b):
            for g in range(4):
                o = pl.ds(128 + g * 16, 16)
                o0 = pl.ds(g * 16, 16)
                ps[b, o0] = ps[b, o]
                pd[b, o0] = pd[b, o]
                pw[b, o0] = pw[b, o]

        def chunk_body(c, carry):
            fills, nfls = carry
            eo = pl.multiple_of(toff + c * c_sz, 128)
            pltpu.sync_copy(src_hbm.at[pl.ds(eo, c_sz)], srcb)
            pltpu.sync_copy(dst_hbm.at[pl.ds(eo, c_sz)], dstb)
            pltpu.sync_copy(ew_hbm.at[pl.ds(eo, c_sz)], ewb)

            def sg_body(sg, carry):
                fills, nfls = carry
                for gi in range(4):
                    o = pl.ds(sg * 64 + gi * 16, 16)
                    dv = dstb[o]
                    sv = srcb[o]
                    wv = ewb[o]
                    new_fills = []
                    for b in range(nb):
                        lo = b * blk
                        m = (dv >= lo) & (dv < lo + blk)
                        cum = plsc.cumsum(m.astype(jnp.int32))
                        pos = fills[b] + cum - 1
                        plsc.store_scatter(ps.at[b], [pos], sv, mask=m)
                        plsc.store_scatter(pd.at[b], [pos], dv - lo,
                                           mask=m)
                        plsc.store_scatter(pw.at[b], [pos], wv, mask=m)
                        new_fills.append(fills[b] + cum[15])
                    fills = tuple(new_fills)
                new_fills, new_nfls = [], []
                for b in range(nb):
                    def do_flush(fn, b=b):
                        f, n = fn
                        flush(b, n)
                        shift(b)
                        return f - 128, n + 1

                    f, n = lax.cond(fills[b] >= 128, do_flush,
                                    lambda fn: fn, (fills[b], nfls[b]))
                    new_fills.append(f)
                    new_nfls.append(n)
                return tuple(new_fills), tuple(new_nfls)

            return lax.fori_loop(0, nsg, sg_body, (fills, nfls))

        zero = jnp.int32(0)
        fills, nfls = lax.fori_loop(
            0, nchunks, chunk_body,
            (tuple(zero for _ in range(nb)), tuple(zero for _ in range(nb))))

        cvec = jnp.zeros((16,), jnp.int32)
        lane16 = lax.iota(jnp.int32, 16)
        for b in range(nb):
            def do_flush2(fn, b=b):
                f, n = fn
                flush(b, n)
                shift(b)
                return f - 128, n + 1

            f, n = lax.cond(fills[b] >= 128, do_flush2, lambda fn: fn,
                            (fills[b], nfls[b]))
            for g in range(8):
                o = pl.ds(g * 16, 16)
                keep = (lane16 + g * 16) < f
                ps[b, o] = jnp.where(keep, ps[b, o], 0)
                pd[b, o] = jnp.where(keep, pd[b, o], 0)
                pw[b, o] = jnp.where(keep, pw[b, o], jnp.float32(0.0))
            flush(b, n)
            cvec = jnp.where(lane16 == b, (n + 1) * 128, cvec)
        cbuf[pl.ds(0, 16)] = cvec
        pltpu.sync_copy(cbuf, cnts.at[pl.ds(w * 16, 16)])

    return bk


@functools.lru_cache(maxsize=None)
def _make_prop(n_rows, d, e_pad, nb, blk, count_mode):
    nw = _NC * _NS
    tw = e_pad // nw
    capr = tw + 128
    rows_pt = blk // _NS
    assert rows_pt % _BATCH == 0 and nb % 2 == 0
    nfl = rows_pt // _BATCH
    out_pad = nb * blk
    nq = d // 16
    mesh = plsc.VectorSubcoreMesh(core_axis_name="c", subcore_axis_name="s",
                                  num_cores=_NC, num_subcores=_NS)

    @functools.partial(
        pl.kernel,
        out_type=jax.ShapeDtypeStruct((out_pad, d), jnp.float32),
        mesh=mesh,
        compiler_params=pltpu.CompilerParams(use_tc_tiling_on_sc=False,
                                             needs_layout_passes=False),
        scratch_types=dict(
            cntv=pltpu.VMEM((nw * 16,), jnp.int32),
            gidx=pltpu.VMEM((_BATCH,), jnp.int32),
            sdst=pltpu.VMEM((_BATCH,), jnp.int32),
            sew=pltpu.VMEM((_BATCH,), jnp.float32),
            rows=pltpu.VMEM((_BATCH, d), jnp.float32),
            zbuf=pltpu.VMEM((_BATCH, d), jnp.float32),
            accum=pltpu.MemorySpace.VMEM_SHARED((blk, d), jnp.float32),
            sem=pltpu.SemaphoreType.DMA,
        ),
    )
    def prop_k(x_hbm, bsrc, bdst, bew, cnts, out_hbm,
               cntv, gidx, sdst, sew, rows, zbuf, accum, sem):
        cid = lax.axis_index("c")
        sid = lax.axis_index("s")
        pltpu.sync_copy(cnts, cntv)
        lane16 = lax.iota(jnp.int32, 16)

        def zb_init(r, carry):
            for q in range(nq):
                zbuf[r, pl.ds(q * 16, 16)] = jnp.zeros((16,), jnp.float32)
            return carry

        lax.fori_loop(0, _BATCH, zb_init, 0)

        def batch(base):
            pltpu.sync_copy(bsrc.at[pl.ds(base, _BATCH)], gidx)
            pltpu.sync_copy(bdst.at[pl.ds(base, _BATCH)], sdst)
            pltpu.sync_copy(bew.at[pl.ds(base, _BATCH)], sew)
            if not count_mode:
                pltpu.async_copy(x_hbm.at[gidx], rows, sem).wait()

            def scale(g, carry):
                wv = sew[pl.ds(g * 16, 16)]
                for i in range(16):
                    w = wv[i]
                    r = g * 16 + i
                    for q in range(nq):
                        o = pl.ds(q * 16, 16)
                        if count_mode:
                            rows[r, o] = jnp.full((16,), 1.0, jnp.float32) * w
                        else:
                            rows[r, o] = rows[r, o] * w
                return carry

            lax.fori_loop(0, _BATCH // 16, scale, 0)
            pltpu.sync_copy(rows, accum.at[sdst], add=True)

        def block_body(bi, carry):
            b = bi * 2 + cid
            lo = b * blk

            def zero(j, c):
                r0 = sid * rows_pt + j * _BATCH
                pltpu.sync_copy(zbuf, accum.at[pl.ds(r0, _BATCH), :])
                return c

            lax.fori_loop(0, nfl, zero, 0)
            plsc.subcore_barrier()

            for wi in range(2):
                w = 2 * sid + wi
                cv = cntv[pl.ds(w * 16, 16)]
                n = jnp.max(jnp.where(lane16 == b, cv, 0)) // _BATCH
                rbase = (w * nb + b) * capr

                def bat(j, c, rbase=rbase):
                    batch(pl.multiple_of(rbase + j * _BATCH, _BATCH))
                    return c

                lax.fori_loop(0, n, bat, 0)
            plsc.subcore_barrier()

            def fl(j, c):
                r0 = sid * rows_pt + j * _BATCH
                pltpu.sync_copy(accum.at[pl.ds(r0, _BATCH), :],
                                out_hbm.at[pl.ds(lo + r0, _BATCH), :])
                return c

            lax.fori_loop(0, nfl, fl, 0)
            plsc.subcore_barrier()
            return carry

        lax.fori_loop(0, nb // 2, block_body, 0)

    return prop_k


def _prop_sc(x, buckets, n_rows, d, nb, blk, count_mode=False):
    bsrc, bdst, bew, cnts = buckets
    nw = _NC * _NS
    capr = bsrc.shape[0] // (nw * nb)
    e_pad = (capr - 128) * nw
    k = _make_prop(n_rows, d, e_pad, nb, blk, count_mode)
    out = k(x, bsrc, bdst, bew, cnts)
    return out[:n_rows]


@functools.lru_cache(maxsize=None)
def _make_gather_pair(n_out, d, e_pad):
    nw = _NC * _NS
    per_w = e_pad // nw
    assert per_w % _BATCH == 0
    nbat = per_w // _BATCH
    nflat = (_BATCH * d) // 16
    mesh = plsc.VectorSubcoreMesh(core_axis_name="c", subcore_axis_name="s",
                                  num_cores=_NC, num_subcores=_NS)

    @functools.partial(
        pl.kernel,
        out_type=jax.ShapeDtypeStruct((e_pad, d), jnp.float32),
        mesh=mesh,
        compiler_params=pltpu.CompilerParams(use_tc_tiling_on_sc=False,
                                             needs_layout_passes=False),
        scratch_types=dict(
            ub=pltpu.VMEM((_BATCH,), jnp.int32),
            vb=pltpu.VMEM((_BATCH,), jnp.int32),
            rowsa=pltpu.VMEM((_BATCH, d), jnp.float32),
            rowsb=pltpu.VMEM((_BATCH, d), jnp.float32),
            sem=pltpu.SemaphoreType.DMA,
            sem2=pltpu.SemaphoreType.DMA,
        ),
    )
    def gather_k(xt_hbm, u_hbm, v_hbm, out_hbm, ub, vb, rowsa, rowsb,
                 sem, sem2):
        cid = lax.axis_index("c")
        sid = lax.axis_index("s")
        wid = sid * _NC + cid
        base = wid * per_w

        def bat(j, carry):
            eo = pl.multiple_of(base + j * _BATCH, _BATCH)
            pltpu.sync_copy(u_hbm.at[pl.ds(eo, _BATCH)], ub)
            pltpu.sync_copy(v_hbm.at[pl.ds(eo, _BATCH)], vb)
            cpa = pltpu.async_copy(xt_hbm.at[ub], rowsa, sem)
            cpb = pltpu.async_copy(xt_hbm.at[vb], rowsb, sem2)
            cpa.wait()
            cpb.wait()

            def addf(g, c):
                r = g // nq_
                o = pl.ds((g % nq_) * 16, 16)
                rowsa[r, o] = 0.5 * (rowsa[r, o] + rowsb[r, o])
                return c

            nq_ = d // 16
            lax.fori_loop(0, nflat, addf, 0)
            pltpu.sync_copy(rowsa, out_hbm.at[pl.ds(eo, _BATCH), :])
            return carry

        lax.fori_loop(0, nbat, bat, 0)

    return gather_k


def _gather_pair_sc(xt, u_p, v_p, n_out, d):
    e_pad = u_p.shape[0]
    k = _make_gather_pair(n_out, d, e_pad)
    return k(xt, u_p, v_p)[:n_out]


def _cbr(x, prop_fn, p, x0=None, colsum=False):
    ws = p['W']
    if len(ws) > 1:
        pr = prop_fn(x)
        w1 = ws[1]
        if pr.shape[1] != w1.shape[0]:
            w1p = jnp.pad(w1, ((0, pr.shape[1] - w1.shape[0]), (0, 0)))
        else:
            w1p = w1
        y, st = _dense([x, pr], [ws[0] + ws[1], -w1p], p['b'], True)
    else:
        y, st = _dense([x], [ws[0]], p['b'], True)
    return _bn_apply(y, st, p['g'], p['bt'], x0=x0, colsum=colsum)


def _lin(x, p):
    y, _ = _dense([x], [p['W'][0]], p['b'], False)
    return y


def kernel(x_s, x_t, edge_index_s, edge_weight_s, edge_index_t,
           edge_weight_t, edge_index, params):
    if _USE_SC_PROP:
        def padE(a, epad, dtype=None):
            return jnp.pad(a, (0, epad - a.shape[0]))

        E_S_PAD, E_T_PAD, E_UV_PAD = 524288, 360448, 327680
        E_G_PAD = 163840
        src_s = padE(edge_index_s[0], E_S_PAD)
        dst_s = padE(edge_index_s[1], E_S_PAD)
        ew_s = padE(edge_weight_s, E_S_PAD)
        src_t = padE(edge_index_t[0], E_T_PAD)
        dst_t = padE(edge_index_t[1], E_T_PAD)
        ew_t = padE(edge_weight_t, E_T_PAD)
        u = edge_index[0]
        v = edge_index[1]
        ar = jnp.arange(N_S, dtype=jnp.int32)
        src_uv = padE(jnp.concatenate([ar, ar]), E_UV_PAD)
        dst_uv = padE(jnp.concatenate([u, v]), E_UV_PAD)
        ew_uv = padE(jnp.ones((2 * N_S,), jnp.float32), E_UV_PAD)
        u_g = padE(u, E_G_PAD)
        v_g = padE(v, E_G_PAD)

        def _padcols(x):
            c = x.shape[1]
            dp = 16 if c < 64 else 64
            return jnp.pad(x, ((0, 0), (0, dp - c))) if c != dp else x, dp

        bk_s = _make_bucketize(E_S_PAD, 10, 16384)(src_s, dst_s, ew_s)
        bk_t = _make_bucketize(E_T_PAD, 2, 6144)(src_t, dst_t, ew_t)
        bk_uv = _make_bucketize(E_UV_PAD, 2, 6144)(src_uv, dst_uv, ew_uv)

        def prop_s(x):
            xp, dp = _padcols(x)
            return _prop_sc(xp, bk_s, N_S, dp, 10, 16384)

        def prop_t(x):
            xp, dp = _padcols(x)
            return _prop_sc(xp, bk_t, N_T, dp, 2, 6144)

        def scatter_uv(xs):
            return _prop_sc(xs, bk_uv, N_T, FEAT, 2, 6144)

        deg64 = _prop_sc(jnp.zeros((8, FEAT), jnp.float32), bk_uv,
                         N_T, FEAT, 2, 6144, count_mode=True)

        def gather_uv(xt):
            return _gather_pair_sc(xt, u_g, v_g, N_S, FEAT)
    else:
        prop_s = lambda x: _prop_jnp(x, edge_index_s[0], edge_index_s[1],
                                     edge_weight_s)
        prop_t = lambda x: _prop_jnp(x, edge_index_t[0], edge_index_t[1],
                                     edge_weight_t)
        u = edge_index[0]
        v = edge_index[1]

        def scatter_uv(xs):
            agg = jnp.zeros((N_T, FEAT), jnp.float32)
            return agg.at[u].add(xs).at[v].add(xs)

        deg64 = scatter_uv(jnp.ones((N_S, FEAT), jnp.float32))

        def gather_uv(xt):
            return 0.5 * (xt[u] + xt[v])

    xs, _ = _cbr(x_s, prop_s, params['HL_EC'])
    xt, _ = _cbr(x_t, prop_t, params['HL_NC'])
    xs0 = xs
    xt0 = xt

    for i in range(4):
        last = i == 3
        for j, p in enumerate(params['NC'][i]):
            xt, cs_t = _cbr(xt, prop_t, p, x0=xt0,
                            colsum=(last and j == 1))
        for j, p in enumerate(params['EC'][i]):
            xs, cs_s = _cbr(xs, prop_s, p, x0=xs0,
                            colsum=(last and j == 1))
        if i < 3:
            agg = scatter_uv(xs)
            temp_xt = _scale_rows(agg, deg64)
            temp_xs = gather_uv(xt)
            xt_c = jnp.concatenate([xt, temp_xt], axis=-1)
            xs_c = jnp.concatenate([xs, temp_xs], axis=-1)
            xt, _ = _cbr(xt_c, prop_t, params['int_e2n'][i][0])
            xt, _ = _cbr(xt, prop_t, params['int_e2n'][i][1])
            xs, _ = _cbr(xs_c, prop_s, params['int_n2e'][i][0])
            xs, _ = _cbr(xs, prop_s, params['int_n2e'][i][1])
            xt0 = _lin(xt0, params['n0_proj'][i])
            xs0 = _lin(xs0, params['e0_proj'][i])

    w_out = params['out']['W']
    b_out = params['out']['b']
    w16 = jnp.pad(w_out, ((0, 0), (0, 16 - w_out.shape[1])))
    b16 = jnp.pad(b_out, (0, 16 - b_out.shape[0])).reshape(1, 16)
    out = _final(cs_s, cs_t, w16, b16)
    return out[0:1, 0:10]


# speedup vs baseline: 1.4755x; 1.4755x over previous
import functools

import jax
import jax.numpy as jnp
from jax import lax
from jax.experimental import pallas as pl
from jax.experimental.pallas import tpu as pltpu
from jax.experimental.pallas import tpu_sc as plsc

N_T = 10000
N_S = 160000
FEAT = 64
ALPHA = 0.5
ROWS = 2000

_USE_SC_PROP = True


def _dense_body(nx, stats, *refs):
    xs = refs[:nx]
    ws = refs[nx:2 * nx]
    b_ref = refs[2 * nx]
    y_ref = refs[2 * nx + 1]
    y = jnp.dot(xs[0][...], ws[0][...], preferred_element_type=jnp.float32)
    for i in range(1, nx):
        y = y + jnp.dot(xs[i][...], ws[i][...],
                        preferred_element_type=jnp.float32)
    y = y + b_ref[...]
    y_ref[...] = y
    if stats:
        st_ref = refs[2 * nx + 2]

        @pl.when(pl.program_id(0) == 0)
        def _():
            st_ref[...] = jnp.zeros_like(st_ref)

        s1 = jnp.sum(y, axis=0, keepdims=True)
        s2 = jnp.sum(y * y, axis=0, keepdims=True)
        st_ref[...] += jnp.concatenate([s1, s2], axis=0)


def _dense(x_list, w_list, b, stats):
    n = x_list[0].shape[0]
    nx = len(x_list)
    cout = w_list[0].shape[1]
    grid = n // ROWS
    in_specs = (
        [pl.BlockSpec((ROWS, x.shape[1]), lambda i: (i, 0)) for x in x_list]
        + [pl.BlockSpec(w.shape, lambda i: (0, 0)) for w in w_list]
        + [pl.BlockSpec((1, cout), lambda i: (0, 0))]
    )
    out_shape = [jax.ShapeDtypeStruct((n, cout), jnp.float32)]
    out_specs = [pl.BlockSpec((ROWS, cout), lambda i: (i, 0))]
    if stats:
        out_shape.append(jax.ShapeDtypeStruct((2, cout), jnp.float32))
        out_specs.append(pl.BlockSpec((2, cout), lambda i: (0, 0)))
    out = pl.pallas_call(
        functools.partial(_dense_body, nx, stats),
        grid=(grid,),
        in_specs=in_specs,
        out_specs=out_specs if stats else out_specs[0],
        out_shape=out_shape if stats else out_shape[0],
    )(*x_list, *w_list, b.reshape(1, cout))
    return out if stats else (out, None)


def _bn_body(n, mix, colsum, *refs):
    y_ref, st_ref, g_ref, bt_ref = refs[:4]
    idx = 4
    if mix:
        x0_ref = refs[idx]
        idx += 1
    o_ref = refs[idx]
    idx += 1
    st = st_ref[...]
    m = st[0:1, :] / n
    v = st[1:2, :] / n - m * m
    scale = g_ref[...] * lax.rsqrt(v + 1e-5)
    o = (y_ref[...] - m) * scale + bt_ref[...]
    o = jnp.maximum(o, 0.0)
    if mix:
        o = (1.0 - ALPHA) * o + ALPHA * x0_ref[...]
    o_ref[...] = o
    if colsum:
        cs_ref = refs[idx]

        @pl.when(pl.program_id(0) == 0)
        def _():
            cs_ref[...] = jnp.zeros_like(cs_ref)

        s1 = jnp.sum(o, axis=0, keepdims=True)
        cs_ref[...] += jnp.concatenate([s1, s1], axis=0)


def _bn_apply(y, st, g, bt, x0=None, colsum=False):
    n, c = y.shape
    grid = n // ROWS
    mix = x0 is not None
    in_specs = [
        pl.BlockSpec((ROWS, c), lambda i: (i, 0)),
        pl.BlockSpec((2, c), lambda i: (0, 0)),
        pl.BlockSpec((1, c), lambda i: (0, 0)),
        pl.BlockSpec((1, c), lambda i: (0, 0)),
    ]
    args = [y, st, g.reshape(1, c), bt.reshape(1, c)]
    if mix:
        in_specs.append(pl.BlockSpec((ROWS, c), lambda i: (i, 0)))
        args.append(x0)
    out_shape = [jax.ShapeDtypeStruct((n, c), jnp.float32)]
    out_specs = [pl.BlockSpec((ROWS, c), lambda i: (i, 0))]
    if colsum:
        out_shape.append(jax.ShapeDtypeStruct((2, c), jnp.float32))
        out_specs.append(pl.BlockSpec((2, c), lambda i: (0, 0)))
    out = pl.pallas_call(
        functools.partial(_bn_body, float(n), mix, colsum),
        grid=(grid,),
        in_specs=in_specs,
        out_specs=out_specs if colsum else out_specs[0],
        out_shape=out_shape if colsum else out_shape[0],
    )(*args)
    return out if colsum else (out, None)


def _scale_rows_body(a_ref, d_ref, o_ref):
    o_ref[...] = a_ref[...] / jnp.maximum(d_ref[...], 1.0)


def _scale_rows(a, d):
    n, c = a.shape
    return pl.pallas_call(
        _scale_rows_body,
        grid=(n // ROWS,),
        in_specs=[pl.BlockSpec((ROWS, c), lambda i: (i, 0)),
                  pl.BlockSpec((ROWS, c), lambda i: (i, 0))],
        out_specs=pl.BlockSpec((ROWS, c), lambda i: (i, 0)),
        out_shape=jax.ShapeDtypeStruct((n, c), jnp.float32),
    )(a, d)


def _final_body(ss_ref, st_ref, w_ref, b_ref, o_ref):
    pooled = jnp.concatenate(
        [ss_ref[0:1, :] / N_S, st_ref[0:1, :] / N_T], axis=1)
    pooled = jnp.broadcast_to(pooled, (8, 2 * FEAT))
    o_ref[...] = jnp.dot(pooled, w_ref[...],
                         preferred_element_type=jnp.float32) + b_ref[...]


def _final(sum_s, sum_t, w16, b16):
    return pl.pallas_call(
        _final_body,
        in_specs=[pl.BlockSpec((2, FEAT), lambda: (0, 0)),
                  pl.BlockSpec((2, FEAT), lambda: (0, 0)),
                  pl.BlockSpec((2 * FEAT, 16), lambda: (0, 0)),
                  pl.BlockSpec((1, 16), lambda: (0, 0))],
        out_specs=pl.BlockSpec((8, 16), lambda: (0, 0)),
        out_shape=jax.ShapeDtypeStruct((8, 16), jnp.float32),
    )(sum_s, sum_t, w16, b16)


def _prop_jnp(x, src, dst, ew):
    msg = ew[:, None] * x[src]
    return jnp.zeros_like(x).at[dst].add(msg)


_NC = 2
_NS = 16


def _take16(x, idx):
    return lax.gather(
        x, idx[:, None],
        lax.GatherDimensionNumbers(offset_dims=(),
                                   collapsed_slice_dims=(0,),
                                   start_index_map=(0,)),
        slice_sizes=(1,),
        mode=lax.GatherScatterMode.PROMISE_IN_BOUNDS)
_CHUNK = 2048
_BATCH = 128


@functools.lru_cache(maxsize=None)
def _make_bucketize(e_pad, nb, blk):
    nw = _NC * _NS
    tw = e_pad // nw
    c_sz = 2048 if tw % 2048 == 0 else 1024
    assert tw % c_sz == 0 and tw % 128 == 0
    nchunks = tw // c_sz
    nsg = c_sz // 64
    capr = tw + 128
    out_sz = nw * nb * capr
    pcap = 192
    mesh = plsc.VectorSubcoreMesh(core_axis_name="c", subcore_axis_name="s",
                                  num_cores=_NC, num_subcores=_NS)

    @functools.partial(
        pl.kernel,
        out_type=(jax.ShapeDtypeStruct((out_sz,), jnp.int32),
                  jax.ShapeDtypeStruct((out_sz,), jnp.int32),
                  jax.ShapeDtypeStruct((out_sz,), jnp.float32),
                  jax.ShapeDtypeStruct((nw * 16,), jnp.int32)),
        mesh=mesh,
        compiler_params=pltpu.CompilerParams(use_tc_tiling_on_sc=False,
                                             needs_layout_passes=False),
        scratch_types=dict(
            srcb=pltpu.VMEM((c_sz,), jnp.int32),
            dstb=pltpu.VMEM((c_sz,), jnp.int32),
            ewb=pltpu.VMEM((c_sz,), jnp.float32),
            ps=pltpu.VMEM((nb, pcap), jnp.int32),
            pd=pltpu.VMEM((nb, pcap), jnp.int32),
            pw=pltpu.VMEM((nb, pcap), jnp.float32),
            cbuf=pltpu.VMEM((16,), jnp.int32),
        ),
    )
    def bk(src_hbm, dst_hbm, ew_hbm, bsrc, bdst, bew, cnts,
           srcb, dstb, ewb, ps, pd, pw, cbuf):
        cid = lax.axis_index("c")
        sid = lax.axis_index("s")
        w = sid * _NC + cid
        toff = w * tw

        def flush(b, nfl):
            base = pl.multiple_of((w * nb + b) * capr + nfl * 128, 128)
            pltpu.sync_copy(ps.at[b, pl.ds(0, 128)],
                            bsrc.at[pl.ds(base, 128)])
            pltpu.sync_copy(pd.at[b, pl.ds(0, 128)],
                            bdst.at[pl.ds(base, 128)])
            pltpu.sync_copy(pw.at[b, pl.ds(0, 128)],
                            bew.at[pl.ds(base, 128)])

        def shift(b):
            for g in range(4):
                o = pl.ds(128 + g * 16, 16)
                o0 = pl.ds(g * 16, 16)
                ps[b, o0] = ps[b, o]
                pd[b, o0] = pd[b, o]
                pw[b, o0] = pw[b, o]

        def chunk_body(c, carry):
            fills, nfls = carry
            eo = pl.multiple_of(toff + c * c_sz, 128)
            pltpu.sync_copy(src_hbm.at[pl.ds(eo, c_sz)], srcb)
            pltpu.sync_copy(dst_hbm.at[pl.ds(eo, c_sz)], dstb)
            pltpu.sync_copy(ew_hbm.at[pl.ds(eo, c_sz)], ewb)

            def sg_body(sg, carry):
                fills, nfls = carry
                last = jnp.full((16,), 15, jnp.int32)
                for gi in range(4):
                    o = pl.ds(sg * 64 + gi * 16, 16)
                    dv = dstb[o]
                    sv = srcb[o]
                    wv = ewb[o]
                    new_fills = []
                    for b in range(nb):
                        lo = b * blk
                        m = (dv >= lo) & (dv < lo + blk)
                        cum = plsc.cumsum(m.astype(jnp.int32))
                        pos = fills[b] + cum - 1
                        plsc.store_scatter(ps.at[b], [pos], sv, mask=m)
                        plsc.store_scatter(pd.at[b], [pos], dv - lo,
                                           mask=m)
                        plsc.store_scatter(pw.at[b], [pos], wv, mask=m)
                        tot = _take16(cum, last)
                        new_fills.append(fills[b] + tot)
                    fills = tuple(new_fills)
                new_fills, new_nfls = [], []
                for b in range(nb):
                    def do_flush(fn, b=b):
                        f, n = fn
                        flush(b, n)
                        shift(b)
                        return f - 128, n + 1

                    f, n = lax.cond(fills[b][0] >= 128, do_flush,
                                    lambda fn: fn, (fills[b], nfls[b]))
                    new_fills.append(f)
                    new_nfls.append(n)
                return tuple(new_fills), tuple(new_nfls)

            return lax.fori_loop(0, nsg, sg_body, (fills, nfls))

        zero_v = jnp.zeros((16,), jnp.int32)
        zero = jnp.int32(0)
        fills, nfls = lax.fori_loop(
            0, nchunks, chunk_body,
            (tuple(zero_v for _ in range(nb)),
             tuple(zero for _ in range(nb))))

        cvec = jnp.zeros((16,), jnp.int32)
        lane16 = lax.iota(jnp.int32, 16)
        for b in range(nb):
            def do_flush2(fn, b=b):
                f, n = fn
                flush(b, n)
                shift(b)
                return f - 128, n + 1

            f, n = lax.cond(fills[b][0] >= 128, do_flush2, lambda fn: fn,
                            (fills[b], nfls[b]))
            for g in range(8):
                o = pl.ds(g * 16, 16)
                keep = (lane16 + g * 16) < f
                ps[b, o] = jnp.where(keep, ps[b, o], 0)
                pd[b, o] = jnp.where(keep, pd[b, o], 0)
                pw[b, o] = jnp.where(keep, pw[b, o], jnp.float32(0.0))
            flush(b, n)
            cvec = jnp.where(lane16 == b, (n + 1) * 128, cvec)
        cbuf[pl.ds(0, 16)] = cvec
        pltpu.sync_copy(cbuf, cnts.at[pl.ds(w * 16, 16)])

    return bk


@functools.lru_cache(maxsize=None)
def _make_prop(n_rows, d, e_pad, nb, blk, count_mode):
    nw = _NC * _NS
    tw = e_pad // nw
    capr = tw + 128
    rows_pt = blk // _NS
    assert rows_pt % _BATCH == 0 and nb % 2 == 0
    nfl = rows_pt // _BATCH
    out_pad = nb * blk
    nq = d // 16
    mesh = plsc.VectorSubcoreMesh(core_axis_name="c", subcore_axis_name="s",
                                  num_cores=_NC, num_subcores=_NS)

    @functools.partial(
        pl.kernel,
        out_type=jax.ShapeDtypeStruct((out_pad, d), jnp.float32),
        mesh=mesh,
        compiler_params=pltpu.CompilerParams(use_tc_tiling_on_sc=False,
                                             needs_layout_passes=False),
        scratch_types=dict(
            cntv=pltpu.VMEM((nw * 16,), jnp.int32),
            srcb2=pltpu.VMEM((2, _BATCH), jnp.int32),
            dstb2=pltpu.VMEM((2, _BATCH), jnp.int32),
            ewb2=pltpu.VMEM((2, _BATCH), jnp.float32),
            sdst=pltpu.VMEM((_BATCH,), jnp.int32),
            rows2=pltpu.VMEM((2, _BATCH, d), jnp.float32),
            zbuf=pltpu.VMEM((_BATCH, d), jnp.float32),
            accum=pltpu.MemorySpace.VMEM_SHARED((blk, d), jnp.float32),
            semr=pltpu.SemaphoreType.DMA((2,)),
            semg=pltpu.SemaphoreType.DMA((2,)),
        ),
    )
    def prop_k(x_hbm, bsrc, bdst, bew, cnts, out_hbm,
               cntv, srcb2, dstb2, ewb2, sdst, rows2, zbuf, accum,
               semr, semg):
        cid = lax.axis_index("c")
        sid = lax.axis_index("s")
        pltpu.sync_copy(cnts, cntv)
        lane16 = lax.iota(jnp.int32, 16)

        def zb_init(r, carry):
            for q in range(nq):
                zbuf[r, pl.ds(q * 16, 16)] = jnp.zeros((16,), jnp.float32)
            return carry

        lax.fori_loop(0, _BATCH, zb_init, 0)

        def rec_issue(rbase, j, p):
            base = pl.multiple_of(rbase + j * _BATCH, _BATCH)
            pltpu.async_copy(bsrc.at[pl.ds(base, _BATCH)], srcb2.at[p],
                             semr.at[p])
            pltpu.async_copy(bdst.at[pl.ds(base, _BATCH)], dstb2.at[p],
                             semr.at[p])
            pltpu.async_copy(bew.at[pl.ds(base, _BATCH)], ewb2.at[p],
                             semr.at[p])

        def rec_wait(rbase, j, p):
            base = pl.multiple_of(rbase + j * _BATCH, _BATCH)
            pltpu.make_async_copy(bsrc.at[pl.ds(base, _BATCH)],
                                  srcb2.at[p], semr.at[p]).wait()
            pltpu.make_async_copy(bdst.at[pl.ds(base, _BATCH)],
                                  dstb2.at[p], semr.at[p]).wait()
            pltpu.make_async_copy(bew.at[pl.ds(base, _BATCH)],
                                  ewb2.at[p], semr.at[p]).wait()

        def g_issue(p):
            if not count_mode:
                pltpu.async_copy(x_hbm.at[srcb2.at[p]], rows2.at[p],
                                 semg.at[p])

        def g_wait(p):
            if not count_mode:
                pltpu.make_async_copy(x_hbm.at[srcb2.at[p]], rows2.at[p],
                                      semg.at[p]).wait()

        def finish(p):
            for g in range(_BATCH // 16):
                o = pl.ds(g * 16, 16)
                sdst[o] = dstb2[p, o]

            def scale(g, carry):
                wv = ewb2[p, pl.ds(g * 16, 16)]
                for i in range(16):
                    wsp = _take16(wv, jnp.full((16,), i, jnp.int32))
                    r = g * 16 + i
                    for q in range(nq):
                        o = pl.ds(q * 16, 16)
                        if count_mode:
                            rows2[p, r, o] = wsp
                        else:
                            rows2[p, r, o] = rows2[p, r, o] * wsp
                return carry

            lax.fori_loop(0, _BATCH // 16, scale, 0)
            pltpu.sync_copy(rows2.at[p], accum.at[sdst], add=True)

        def region(rbase, n):
            rec_issue(rbase, 0, 0)
            rec_wait(rbase, 0, 0)
            g_issue(0)
            rec_issue(rbase, 1, 1)

            def it(j, c):
                p = j & 1

                def start_next(_):
                    rec_wait(rbase, j + 1, 1 - p)
                    g_issue(1 - p)
                    return 0

                lax.cond(j + 1 < n, start_next, lambda _: 0, 0)
                g_wait(p)
                finish(p)
                rec_issue(rbase, j + 2, p)
                return c

            lax.fori_loop(0, n, it, 0)
            rec_wait(rbase, n, n & 1)
            rec_wait(rbase, n + 1, (n + 1) & 1)

        def block_body(bi, carry):
            b = bi * 2 + cid
            lo = b * blk

            def zero(j, c):
                r0 = sid * rows_pt + j * _BATCH
                pltpu.sync_copy(zbuf, accum.at[pl.ds(r0, _BATCH), :])
                return c

            lax.fori_loop(0, nfl, zero, 0)
            plsc.subcore_barrier()

            for wi in range(2):
                w = 2 * sid + wi
                cv = cntv[pl.ds(w * 16, 16)]
                n = jnp.max(jnp.where(lane16 == b, cv, 0)) // _BATCH
                region((w * nb + b) * capr, n)
            plsc.subcore_barrier()

            def fl(j, c):
                r0 = sid * rows_pt + j * _BATCH
                pltpu.sync_copy(accum.at[pl.ds(r0, _BATCH), :],
                                out_hbm.at[pl.ds(lo + r0, _BATCH), :])
                return c

            lax.fori_loop(0, nfl, fl, 0)
            plsc.subcore_barrier()
            return carry

        lax.fori_loop(0, nb // 2, block_body, 0)

    return prop_k


def _prop_sc(x, buckets, n_rows, d, nb, blk, count_mode=False):
    bsrc, bdst, bew, cnts = buckets
    nw = _NC * _NS
    capr = bsrc.shape[0] // (nw * nb)
    e_pad = (capr - 128) * nw
    k = _make_prop(n_rows, d, e_pad, nb, blk, count_mode)
    out = k(x, bsrc, bdst, bew, cnts)
    return out[:n_rows]


@functools.lru_cache(maxsize=None)
def _make_gather_pair(n_out, d, e_pad):
    nw = _NC * _NS
    per_w = e_pad // nw
    assert per_w % _BATCH == 0
    nbat = per_w // _BATCH
    nflat = (_BATCH * d) // 16
    mesh = plsc.VectorSubcoreMesh(core_axis_name="c", subcore_axis_name="s",
                                  num_cores=_NC, num_subcores=_NS)

    @functools.partial(
        pl.kernel,
        out_type=jax.ShapeDtypeStruct((e_pad, d), jnp.float32),
        mesh=mesh,
        compiler_params=pltpu.CompilerParams(use_tc_tiling_on_sc=False,
                                             needs_layout_passes=False),
        scratch_types=dict(
            ub=pltpu.VMEM((_BATCH,), jnp.int32),
            vb=pltpu.VMEM((_BATCH,), jnp.int32),
            rowsa=pltpu.VMEM((_BATCH, d), jnp.float32),
            rowsb=pltpu.VMEM((_BATCH, d), jnp.float32),
            sem=pltpu.SemaphoreType.DMA,
            sem2=pltpu.SemaphoreType.DMA,
        ),
    )
    def gather_k(xt_hbm, u_hbm, v_hbm, out_hbm, ub, vb, rowsa, rowsb,
                 sem, sem2):
        cid = lax.axis_index("c")
        sid = lax.axis_index("s")
        wid = sid * _NC + cid
        base = wid * per_w

        def bat(j, carry):
            eo = pl.multiple_of(base + j * _BATCH, _BATCH)
            pltpu.sync_copy(u_hbm.at[pl.ds(eo, _BATCH)], ub)
            pltpu.sync_copy(v_hbm.at[pl.ds(eo, _BATCH)], vb)
            cpa = pltpu.async_copy(xt_hbm.at[ub], rowsa, sem)
            cpb = pltpu.async_copy(xt_hbm.at[vb], rowsb, sem2)
            cpa.wait()
            cpb.wait()

            def addf(g, c):
                r = g // nq_
                o = pl.ds((g % nq_) * 16, 16)
                rowsa[r, o] = 0.5 * (rowsa[r, o] + rowsb[r, o])
                return c

            nq_ = d // 16
            lax.fori_loop(0, nflat, addf, 0)
            pltpu.sync_copy(rowsa, out_hbm.at[pl.ds(eo, _BATCH), :])
            return carry

        lax.fori_loop(0, nbat, bat, 0)

    return gather_k


def _gather_pair_sc(xt, u_p, v_p, n_out, d):
    e_pad = u_p.shape[0]
    k = _make_gather_pair(n_out, d, e_pad)
    return k(xt, u_p, v_p)[:n_out]


def _cbr(x, prop_fn, p, x0=None, colsum=False):
    ws = p['W']
    if len(ws) > 1:
        pr = prop_fn(x)
        w1 = ws[1]
        if pr.shape[1] != w1.shape[0]:
            w1p = jnp.pad(w1, ((0, pr.shape[1] - w1.shape[0]), (0, 0)))
        else:
            w1p = w1
        y, st = _dense([x, pr], [ws[0] + ws[1], -w1p], p['b'], True)
    else:
        y, st = _dense([x], [ws[0]], p['b'], True)
    return _bn_apply(y, st, p['g'], p['bt'], x0=x0, colsum=colsum)


def _lin(x, p):
    y, _ = _dense([x], [p['W'][0]], p['b'], False)
    return y


def kernel(x_s, x_t, edge_index_s, edge_weight_s, edge_index_t,
           edge_weight_t, edge_index, params):
    if _USE_SC_PROP:
        def padE(a, epad, dtype=None):
            return jnp.pad(a, (0, epad - a.shape[0]))

        E_S_PAD, E_T_PAD, E_UV_PAD = 524288, 360448, 327680
        E_G_PAD = 163840
        src_s = padE(edge_index_s[0], E_S_PAD)
        dst_s = padE(edge_index_s[1], E_S_PAD)
        ew_s = padE(edge_weight_s, E_S_PAD)
        src_t = padE(edge_index_t[0], E_T_PAD)
        dst_t = padE(edge_index_t[1], E_T_PAD)
        ew_t = padE(edge_weight_t, E_T_PAD)
        u = edge_index[0]
        v = edge_index[1]
        ar = jnp.arange(N_S, dtype=jnp.int32)
        src_uv = padE(jnp.concatenate([ar, ar]), E_UV_PAD)
        dst_uv = padE(jnp.concatenate([u, v]), E_UV_PAD)
        ew_uv = padE(jnp.ones((2 * N_S,), jnp.float32), E_UV_PAD)
        u_g = padE(u, E_G_PAD)
        v_g = padE(v, E_G_PAD)

        def _padcols(x):
            c = x.shape[1]
            dp = 16 if c < 64 else 64
            return jnp.pad(x, ((0, 0), (0, dp - c))) if c != dp else x, dp

        bk_s = _make_bucketize(E_S_PAD, 10, 16384)(src_s, dst_s, ew_s)
        bk_t = _make_bucketize(E_T_PAD, 2, 6144)(src_t, dst_t, ew_t)
        bk_uv = _make_bucketize(E_UV_PAD, 2, 6144)(src_uv, dst_uv, ew_uv)

        def prop_s(x):
            xp, dp = _padcols(x)
            return _prop_sc(xp, bk_s, N_S, dp, 10, 16384)

        def prop_t(x):
            xp, dp = _padcols(x)
            return _prop_sc(xp, bk_t, N_T, dp, 2, 6144)

        def scatter_uv(xs):
            return _prop_sc(xs, bk_uv, N_T, FEAT, 2, 6144)

        deg64 = _prop_sc(jnp.zeros((8, FEAT), jnp.float32), bk_uv,
                         N_T, FEAT, 2, 6144, count_mode=True)

        def gather_uv(xt):
            return _gather_pair_sc(xt, u_g, v_g, N_S, FEAT)
    else:
        prop_s = lambda x: _prop_jnp(x, edge_index_s[0], edge_index_s[1],
                                     edge_weight_s)
        prop_t = lambda x: _prop_jnp(x, edge_index_t[0], edge_index_t[1],
                                     edge_weight_t)
        u = edge_index[0]
        v = edge_index[1]

        def scatter_uv(xs):
            agg = jnp.zeros((N_T, FEAT), jnp.float32)
            return agg.at[u].add(xs).at[v].add(xs)

        deg64 = scatter_uv(jnp.ones((N_S, FEAT), jnp.float32))

        def gather_uv(xt):
            return 0.5 * (xt[u] + xt[v])

    xs, _ = _cbr(x_s, prop_s, params['HL_EC'])
    xt, _ = _cbr(x_t, prop_t, params['HL_NC'])
    xs0 = xs
    xt0 = xt

    for i in range(4):
        last = i == 3
        for j, p in enumerate(params['NC'][i]):
            xt, cs_t = _cbr(xt, prop_t, p, x0=xt0,
                            colsum=(last and j == 1))
        for j, p in enumerate(params['EC'][i]):
            xs, cs_s = _cbr(xs, prop_s, p, x0=xs0,
                            colsum=(last and j == 1))
        if i < 3:
            agg = scatter_uv(xs)
            temp_xt = _scale_rows(agg, deg64)
            temp_xs = gather_uv(xt)
            xt_c = jnp.concatenate([xt, temp_xt], axis=-1)
            xs_c = jnp.concatenate([xs, temp_xs], axis=-1)
            xt, _ = _cbr(xt_c, prop_t, params['int_e2n'][i][0])
            xt, _ = _cbr(xt, prop_t, params['int_e2n'][i][1])
            xs, _ = _cbr(xs_c, prop_s, params['int_n2e'][i][0])
            xs, _ = _cbr(xs, prop_s, params['int_n2e'][i][1])
            xt0 = _lin(xt0, params['n0_proj'][i])
            xs0 = _lin(xs0, params['e0_proj'][i])

    w_out = params['out']['W']
    b_out = params['out']['b']
    w16 = jnp.pad(w_out, ((0, 0), (0, 16 - w_out.shape[1])))
    b16 = jnp.pad(b_out, (0, 16 - b_out.shape[0])).reshape(1, 16)
    out = _final(cs_s, cs_t, w16, b16)
    return out[0:1, 0:10]


# speedup vs baseline: 1.5432x; 1.0459x over previous
import functools

import jax
import jax.numpy as jnp
from jax import lax
from jax.experimental import pallas as pl
from jax.experimental.pallas import tpu as pltpu
from jax.experimental.pallas import tpu_sc as plsc

N_T = 10000
N_S = 160000
FEAT = 64
ALPHA = 0.5
ROWS = 2000

_USE_SC_PROP = True


def _dense_body(nx, stats, *refs):
    xs = refs[:nx]
    ws = refs[nx:2 * nx]
    b_ref = refs[2 * nx]
    y_ref = refs[2 * nx + 1]
    y = jnp.dot(xs[0][...], ws[0][...], preferred_element_type=jnp.float32)
    for i in range(1, nx):
        y = y + jnp.dot(xs[i][...], ws[i][...],
                        preferred_element_type=jnp.float32)
    y = y + b_ref[...]
    y_ref[...] = y
    if stats:
        st_ref = refs[2 * nx + 2]

        @pl.when(pl.program_id(0) == 0)
        def _():
            st_ref[...] = jnp.zeros_like(st_ref)

        s1 = jnp.sum(y, axis=0, keepdims=True)
        s2 = jnp.sum(y * y, axis=0, keepdims=True)
        st_ref[...] += jnp.concatenate([s1, s2], axis=0)


def _dense(x_list, w_list, b, stats):
    n = x_list[0].shape[0]
    nx = len(x_list)
    cout = w_list[0].shape[1]
    grid = n // ROWS
    in_specs = (
        [pl.BlockSpec((ROWS, x.shape[1]), lambda i: (i, 0)) for x in x_list]
        + [pl.BlockSpec(w.shape, lambda i: (0, 0)) for w in w_list]
        + [pl.BlockSpec((1, cout), lambda i: (0, 0))]
    )
    out_shape = [jax.ShapeDtypeStruct((n, cout), jnp.float32)]
    out_specs = [pl.BlockSpec((ROWS, cout), lambda i: (i, 0))]
    if stats:
        out_shape.append(jax.ShapeDtypeStruct((2, cout), jnp.float32))
        out_specs.append(pl.BlockSpec((2, cout), lambda i: (0, 0)))
    out = pl.pallas_call(
        functools.partial(_dense_body, nx, stats),
        grid=(grid,),
        in_specs=in_specs,
        out_specs=out_specs if stats else out_specs[0],
        out_shape=out_shape if stats else out_shape[0],
    )(*x_list, *w_list, b.reshape(1, cout))
    return out if stats else (out, None)


def _bn_body(n, mix, colsum, *refs):
    y_ref, st_ref, g_ref, bt_ref = refs[:4]
    idx = 4
    if mix:
        x0_ref = refs[idx]
        idx += 1
    o_ref = refs[idx]
    idx += 1
    st = st_ref[...]
    m = st[0:1, :] / n
    v = st[1:2, :] / n - m * m
    scale = g_ref[...] * lax.rsqrt(v + 1e-5)
    o = (y_ref[...] - m) * scale + bt_ref[...]
    o = jnp.maximum(o, 0.0)
    if mix:
        o = (1.0 - ALPHA) * o + ALPHA * x0_ref[...]
    o_ref[...] = o
    if colsum:
        cs_ref = refs[idx]

        @pl.when(pl.program_id(0) == 0)
        def _():
            cs_ref[...] = jnp.zeros_like(cs_ref)

        s1 = jnp.sum(o, axis=0, keepdims=True)
        cs_ref[...] += jnp.concatenate([s1, s1], axis=0)


def _bn_apply(y, st, g, bt, x0=None, colsum=False):
    n, c = y.shape
    grid = n // ROWS
    mix = x0 is not None
    in_specs = [
        pl.BlockSpec((ROWS, c), lambda i: (i, 0)),
        pl.BlockSpec((2, c), lambda i: (0, 0)),
        pl.BlockSpec((1, c), lambda i: (0, 0)),
        pl.BlockSpec((1, c), lambda i: (0, 0)),
    ]
    args = [y, st, g.reshape(1, c), bt.reshape(1, c)]
    if mix:
        in_specs.append(pl.BlockSpec((ROWS, c), lambda i: (i, 0)))
        args.append(x0)
    out_shape = [jax.ShapeDtypeStruct((n, c), jnp.float32)]
    out_specs = [pl.BlockSpec((ROWS, c), lambda i: (i, 0))]
    if colsum:
        out_shape.append(jax.ShapeDtypeStruct((2, c), jnp.float32))
        out_specs.append(pl.BlockSpec((2, c), lambda i: (0, 0)))
    out = pl.pallas_call(
        functools.partial(_bn_body, float(n), mix, colsum),
        grid=(grid,),
        in_specs=in_specs,
        out_specs=out_specs if colsum else out_specs[0],
        out_shape=out_shape if colsum else out_shape[0],
    )(*args)
    return out if colsum else (out, None)


def _scale_rows_body(a_ref, d_ref, o_ref):
    o_ref[...] = a_ref[...] / jnp.maximum(d_ref[...], 1.0)


def _scale_rows(a, d):
    n, c = a.shape
    return pl.pallas_call(
        _scale_rows_body,
        grid=(n // ROWS,),
        in_specs=[pl.BlockSpec((ROWS, c), lambda i: (i, 0)),
                  pl.BlockSpec((ROWS, c), lambda i: (i, 0))],
        out_specs=pl.BlockSpec((ROWS, c), lambda i: (i, 0)),
        out_shape=jax.ShapeDtypeStruct((n, c), jnp.float32),
    )(a, d)


def _final_body(ss_ref, st_ref, w_ref, b_ref, o_ref):
    pooled = jnp.concatenate(
        [ss_ref[0:1, :] / N_S, st_ref[0:1, :] / N_T], axis=1)
    pooled = jnp.broadcast_to(pooled, (8, 2 * FEAT))
    o_ref[...] = jnp.dot(pooled, w_ref[...],
                         preferred_element_type=jnp.float32) + b_ref[...]


def _final(sum_s, sum_t, w16, b16):
    return pl.pallas_call(
        _final_body,
        in_specs=[pl.BlockSpec((2, FEAT), lambda: (0, 0)),
                  pl.BlockSpec((2, FEAT), lambda: (0, 0)),
                  pl.BlockSpec((2 * FEAT, 16), lambda: (0, 0)),
                  pl.BlockSpec((1, 16), lambda: (0, 0))],
        out_specs=pl.BlockSpec((8, 16), lambda: (0, 0)),
        out_shape=jax.ShapeDtypeStruct((8, 16), jnp.float32),
    )(sum_s, sum_t, w16, b16)


def _prop_jnp(x, src, dst, ew):
    msg = ew[:, None] * x[src]
    return jnp.zeros_like(x).at[dst].add(msg)


_NC = 2
_NS = 16


def _take16(x, idx):
    return lax.gather(
        x, idx[:, None],
        lax.GatherDimensionNumbers(offset_dims=(),
                                   collapsed_slice_dims=(0,),
                                   start_index_map=(0,)),
        slice_sizes=(1,),
        mode=lax.GatherScatterMode.PROMISE_IN_BOUNDS)
_CHUNK = 2048
_BATCH = 128


@functools.lru_cache(maxsize=None)
def _make_bucketize(e_pad, nb, blk):
    nw = _NC * _NS
    tw = e_pad // nw
    c_sz = 2048 if tw % 2048 == 0 else 1024
    assert tw % c_sz == 0 and tw % 128 == 0
    nchunks = tw // c_sz
    nsg = c_sz // 64
    capr = tw + 128
    out_sz = nw * nb * capr + 512
    pcap = 192
    mesh = plsc.VectorSubcoreMesh(core_axis_name="c", subcore_axis_name="s",
                                  num_cores=_NC, num_subcores=_NS)

    @functools.partial(
        pl.kernel,
        out_type=(jax.ShapeDtypeStruct((out_sz,), jnp.int32),
                  jax.ShapeDtypeStruct((out_sz,), jnp.int32),
                  jax.ShapeDtypeStruct((out_sz,), jnp.float32),
                  jax.ShapeDtypeStruct((nw * 16,), jnp.int32)),
        mesh=mesh,
        compiler_params=pltpu.CompilerParams(use_tc_tiling_on_sc=False,
                                             needs_layout_passes=False),
        scratch_types=dict(
            srcb=pltpu.VMEM((c_sz,), jnp.int32),
            dstb=pltpu.VMEM((c_sz,), jnp.int32),
            ewb=pltpu.VMEM((c_sz,), jnp.float32),
            ps=pltpu.VMEM((nb, pcap), jnp.int32),
            pd=pltpu.VMEM((nb, pcap), jnp.int32),
            pw=pltpu.VMEM((nb, pcap), jnp.float32),
            cbuf=pltpu.VMEM((16,), jnp.int32),
        ),
    )
    def bk(src_hbm, dst_hbm, ew_hbm, bsrc, bdst, bew, cnts,
           srcb, dstb, ewb, ps, pd, pw, cbuf):
        cid = lax.axis_index("c")
        sid = lax.axis_index("s")
        w = sid * _NC + cid
        toff = w * tw

        def flush(b, nfl):
            base = pl.multiple_of((w * nb + b) * capr + nfl * 128, 128)
            pltpu.sync_copy(ps.at[b, pl.ds(0, 128)],
                            bsrc.at[pl.ds(base, 128)])
            pltpu.sync_copy(pd.at[b, pl.ds(0, 128)],
                            bdst.at[pl.ds(base, 128)])
            pltpu.sync_copy(pw.at[b, pl.ds(0, 128)],
                            bew.at[pl.ds(base, 128)])

        def shift(b):
            for g in range(4):
                o = pl.ds(128 + g * 16, 16)
                o0 = pl.ds(g * 16, 16)
                ps[b, o0] = ps[b, o]
                pd[b, o0] = pd[b, o]
                pw[b, o0] = pw[b, o]

        def chunk_body(c, carry):
            fills, nfls = carry
            eo = pl.multiple_of(toff + c * c_sz, 128)
            pltpu.sync_copy(src_hbm.at[pl.ds(eo, c_sz)], srcb)
            pltpu.sync_copy(dst_hbm.at[pl.ds(eo, c_sz)], dstb)
            pltpu.sync_copy(ew_hbm.at[pl.ds(eo, c_sz)], ewb)

            def sg_body(sg, carry):
                fills, nfls = carry
                last = jnp.full((16,), 15, jnp.int32)
                for gi in range(4):
                    o = pl.ds(sg * 64 + gi * 16, 16)
                    dv = dstb[o]
                    sv = srcb[o]
                    wv = ewb[o]
                    new_fills = []
                    for b in range(nb):
                        lo = b * blk
                        m = (dv >= lo) & (dv < lo + blk)
                        cum = plsc.cumsum(m.astype(jnp.int32))
                        pos = fills[b] + cum - 1
                        plsc.store_scatter(ps.at[b], [pos], sv, mask=m)
                        plsc.store_scatter(pd.at[b], [pos], dv - lo,
                                           mask=m)
                        plsc.store_scatter(pw.at[b], [pos], wv, mask=m)
                        tot = _take16(cum, last)
                        new_fills.append(fills[b] + tot)
                    fills = tuple(new_fills)
                new_fills, new_nfls = [], []
                for b in range(nb):
                    def do_flush(fn, b=b):
                        f, n = fn
                        flush(b, n)
                        shift(b)
                        return f - 128, n + 1

                    f, n = lax.cond(fills[b][0] >= 128, do_flush,
                                    lambda fn: fn, (fills[b], nfls[b]))
                    new_fills.append(f)
                    new_nfls.append(n)
                return tuple(new_fills), tuple(new_nfls)

            return lax.fori_loop(0, nsg, sg_body, (fills, nfls))

        zero_v = jnp.zeros((16,), jnp.int32)
        zero = jnp.int32(0)
        fills, nfls = lax.fori_loop(
            0, nchunks, chunk_body,
            (tuple(zero_v for _ in range(nb)),
             tuple(zero for _ in range(nb))))

        cvec = jnp.zeros((16,), jnp.int32)
        lane16 = lax.iota(jnp.int32, 16)
        for b in range(nb):
            def do_flush2(fn, b=b):
                f, n = fn
                flush(b, n)
                shift(b)
                return f - 128, n + 1

            f, n = lax.cond(fills[b][0] >= 128, do_flush2, lambda fn: fn,
                            (fills[b], nfls[b]))
            for g in range(8):
                o = pl.ds(g * 16, 16)
                keep = (lane16 + g * 16) < f
                ps[b, o] = jnp.where(keep, ps[b, o], 0)
                pd[b, o] = jnp.where(keep, pd[b, o], 0)
                pw[b, o] = jnp.where(keep, pw[b, o], jnp.float32(0.0))
            flush(b, n)
            cvec = jnp.where(lane16 == b, (n + 1) * 128, cvec)
        cbuf[pl.ds(0, 16)] = cvec
        pltpu.sync_copy(cbuf, cnts.at[pl.ds(w * 16, 16)])

    return bk


@functools.lru_cache(maxsize=None)
def _make_prop(n_rows, d, e_pad, nb, blk, count_mode):
    nw = _NC * _NS
    tw = e_pad // nw
    capr = tw + 128
    rows_pt = blk // _NS
    assert rows_pt % _BATCH == 0 and nb % 2 == 0
    nfl = rows_pt // _BATCH
    out_pad = nb * blk
    nq = d // 16
    mesh = plsc.VectorSubcoreMesh(core_axis_name="c", subcore_axis_name="s",
                                  num_cores=_NC, num_subcores=_NS)

    @functools.partial(
        pl.kernel,
        out_type=jax.ShapeDtypeStruct((out_pad, d), jnp.float32),
        mesh=mesh,
        compiler_params=pltpu.CompilerParams(use_tc_tiling_on_sc=False,
                                             needs_layout_passes=False),
        scratch_types=dict(
            cntv=pltpu.VMEM((nw * 16,), jnp.int32),
            srcb2=pltpu.VMEM((4, _BATCH), jnp.int32),
            dstb2=pltpu.VMEM((4, _BATCH), jnp.int32),
            ewb2=pltpu.VMEM((4, _BATCH), jnp.float32),
            sdst=pltpu.VMEM((_BATCH,), jnp.int32),
            rows2=pltpu.VMEM((4, _BATCH, d), jnp.float32),
            zbuf=pltpu.VMEM((_BATCH, d), jnp.float32),
            accum=pltpu.MemorySpace.VMEM_SHARED((blk, d), jnp.float32),
            semr=pltpu.SemaphoreType.DMA((4,)),
            semg=pltpu.SemaphoreType.DMA((4,)),
        ),
    )
    def prop_k(x_hbm, bsrc, bdst, bew, cnts, out_hbm,
               cntv, srcb2, dstb2, ewb2, sdst, rows2, zbuf, accum,
               semr, semg):
        cid = lax.axis_index("c")
        sid = lax.axis_index("s")
        pltpu.sync_copy(cnts, cntv)
        lane16 = lax.iota(jnp.int32, 16)

        def zb_init(r, carry):
            for q in range(nq):
                zbuf[r, pl.ds(q * 16, 16)] = jnp.zeros((16,), jnp.float32)
            return carry

        lax.fori_loop(0, _BATCH, zb_init, 0)

        def rec_issue(rbase, j, p):
            base = pl.multiple_of(rbase + j * _BATCH, _BATCH)
            pltpu.async_copy(bsrc.at[pl.ds(base, _BATCH)], srcb2.at[p],
                             semr.at[p])
            pltpu.async_copy(bdst.at[pl.ds(base, _BATCH)], dstb2.at[p],
                             semr.at[p])
            pltpu.async_copy(bew.at[pl.ds(base, _BATCH)], ewb2.at[p],
                             semr.at[p])

        def rec_wait(rbase, j, p):
            base = pl.multiple_of(rbase + j * _BATCH, _BATCH)
            pltpu.make_async_copy(bsrc.at[pl.ds(base, _BATCH)],
                                  srcb2.at[p], semr.at[p]).wait()
            pltpu.make_async_copy(bdst.at[pl.ds(base, _BATCH)],
                                  dstb2.at[p], semr.at[p]).wait()
            pltpu.make_async_copy(bew.at[pl.ds(base, _BATCH)],
                                  ewb2.at[p], semr.at[p]).wait()

        def g_issue(p):
            if not count_mode:
                pltpu.async_copy(x_hbm.at[srcb2.at[p]], rows2.at[p],
                                 semg.at[p])

        def g_wait(p):
            if not count_mode:
                pltpu.make_async_copy(x_hbm.at[srcb2.at[p]], rows2.at[p],
                                      semg.at[p]).wait()

        def finish(p):
            for g in range(_BATCH // 16):
                o = pl.ds(g * 16, 16)
                sdst[o] = dstb2[p, o]

            def scale(g, carry):
                wv = ewb2[p, pl.ds(g * 16, 16)]
                for i in range(16):
                    wsp = _take16(wv, jnp.full((16,), i, jnp.int32))
                    r = g * 16 + i
                    for q in range(nq):
                        o = pl.ds(q * 16, 16)
                        if count_mode:
                            rows2[p, r, o] = wsp
                        else:
                            rows2[p, r, o] = rows2[p, r, o] * wsp
                return carry

            lax.fori_loop(0, _BATCH // 16, scale, 0)
            pltpu.sync_copy(rows2.at[p], accum.at[sdst], add=True)

        def region(rbase, n):
            for k in range(4):
                rec_issue(rbase, k, k)
            for k in range(3):
                def pro(_, k=k):
                    rec_wait(rbase, k, k)
                    g_issue(k)
                    return 0

                lax.cond(k < n, pro, lambda _: 0, 0)

            def it(j, c):
                def start_next(_):
                    rec_wait(rbase, j + 3, (j + 3) % 4)
                    g_issue((j + 3) % 4)
                    return 0

                lax.cond(j + 3 < n, start_next, lambda _: 0, 0)
                p = j % 4
                g_wait(p)
                finish(p)
                rec_issue(rbase, j + 4, p)
                return c

            lax.fori_loop(0, n, it, 0)
            for k in range(4):
                rec_wait(rbase, n + k, (n + k) % 4)

        def block_body(bi, carry):
            b = bi * 2 + cid
            lo = b * blk

            def zero(j, c):
                r0 = sid * rows_pt + j * _BATCH
                pltpu.sync_copy(zbuf, accum.at[pl.ds(r0, _BATCH), :])
                return c

            lax.fori_loop(0, nfl, zero, 0)
            plsc.subcore_barrier()

            for wi in range(2):
                w = 2 * sid + wi
                cv = cntv[pl.ds(w * 16, 16)]
                n = jnp.max(jnp.where(lane16 == b, cv, 0)) // _BATCH
                region((w * nb + b) * capr, n)
            plsc.subcore_barrier()

            def fl(j, c):
                r0 = sid * rows_pt + j * _BATCH
                pltpu.sync_copy(accum.at[pl.ds(r0, _BATCH), :],
                                out_hbm.at[pl.ds(lo + r0, _BATCH), :])
                return c

            lax.fori_loop(0, nfl, fl, 0)
            plsc.subcore_barrier()
            return carry

        lax.fori_loop(0, nb // 2, block_body, 0)

    return prop_k


def _prop_sc(x, buckets, n_rows, d, nb, blk, count_mode=False):
    bsrc, bdst, bew, cnts = buckets
    nw = _NC * _NS
    capr = (bsrc.shape[0] - 512) // (nw * nb)
    e_pad = (capr - 128) * nw
    k = _make_prop(n_rows, d, e_pad, nb, blk, count_mode)
    out = k(x, bsrc, bdst, bew, cnts)
    return out[:n_rows]


@functools.lru_cache(maxsize=None)
def _make_gather_pair(n_out, d, e_pad):
    nw = _NC * _NS
    per_w = e_pad // nw
    assert per_w % _BATCH == 0
    nbat = per_w // _BATCH
    nflat = (_BATCH * d) // 16
    mesh = plsc.VectorSubcoreMesh(core_axis_name="c", subcore_axis_name="s",
                                  num_cores=_NC, num_subcores=_NS)

    @functools.partial(
        pl.kernel,
        out_type=jax.ShapeDtypeStruct((e_pad, d), jnp.float32),
        mesh=mesh,
        compiler_params=pltpu.CompilerParams(use_tc_tiling_on_sc=False,
                                             needs_layout_passes=False),
        scratch_types=dict(
            ub=pltpu.VMEM((_BATCH,), jnp.int32),
            vb=pltpu.VMEM((_BATCH,), jnp.int32),
            rowsa=pltpu.VMEM((_BATCH, d), jnp.float32),
            rowsb=pltpu.VMEM((_BATCH, d), jnp.float32),
            sem=pltpu.SemaphoreType.DMA,
            sem2=pltpu.SemaphoreType.DMA,
        ),
    )
    def gather_k(xt_hbm, u_hbm, v_hbm, out_hbm, ub, vb, rowsa, rowsb,
                 sem, sem2):
        cid = lax.axis_index("c")
        sid = lax.axis_index("s")
        wid = sid * _NC + cid
        base = wid * per_w

        def bat(j, carry):
            eo = pl.multiple_of(base + j * _BATCH, _BATCH)
            pltpu.sync_copy(u_hbm.at[pl.ds(eo, _BATCH)], ub)
            pltpu.sync_copy(v_hbm.at[pl.ds(eo, _BATCH)], vb)
            cpa = pltpu.async_copy(xt_hbm.at[ub], rowsa, sem)
            cpb = pltpu.async_copy(xt_hbm.at[vb], rowsb, sem2)
            cpa.wait()
            cpb.wait()

            def addf(g, c):
                r = g // nq_
                o = pl.ds((g % nq_) * 16, 16)
                rowsa[r, o] = 0.5 * (rowsa[r, o] + rowsb[r, o])
                return c

            nq_ = d // 16
            lax.fori_loop(0, nflat, addf, 0)
            pltpu.sync_copy(rowsa, out_hbm.at[pl.ds(eo, _BATCH), :])
            return carry

        lax.fori_loop(0, nbat, bat, 0)

    return gather_k


def _gather_pair_sc(xt, u_p, v_p, n_out, d):
    e_pad = u_p.shape[0]
    k = _make_gather_pair(n_out, d, e_pad)
    return k(xt, u_p, v_p)[:n_out]


def _cbr(x, prop_fn, p, x0=None, colsum=False):
    ws = p['W']
    if len(ws) > 1:
        pr = prop_fn(x)
        w1 = ws[1]
        if pr.shape[1] != w1.shape[0]:
            w1p = jnp.pad(w1, ((0, pr.shape[1] - w1.shape[0]), (0, 0)))
        else:
            w1p = w1
        y, st = _dense([x, pr], [ws[0] + ws[1], -w1p], p['b'], True)
    else:
        y, st = _dense([x], [ws[0]], p['b'], True)
    return _bn_apply(y, st, p['g'], p['bt'], x0=x0, colsum=colsum)


def _lin(x, p):
    y, _ = _dense([x], [p['W'][0]], p['b'], False)
    return y


def kernel(x_s, x_t, edge_index_s, edge_weight_s, edge_index_t,
           edge_weight_t, edge_index, params):
    if _USE_SC_PROP:
        def padE(a, epad, dtype=None):
            return jnp.pad(a, (0, epad - a.shape[0]))

        E_S_PAD, E_T_PAD, E_UV_PAD = 524288, 360448, 327680
        E_G_PAD = 163840
        src_s = padE(edge_index_s[0], E_S_PAD)
        dst_s = padE(edge_index_s[1], E_S_PAD)
        ew_s = padE(edge_weight_s, E_S_PAD)
        src_t = padE(edge_index_t[0], E_T_PAD)
        dst_t = padE(edge_index_t[1], E_T_PAD)
        ew_t = padE(edge_weight_t, E_T_PAD)
        u = edge_index[0]
        v = edge_index[1]
        ar = jnp.arange(N_S, dtype=jnp.int32)
        src_uv = padE(jnp.concatenate([ar, ar]), E_UV_PAD)
        dst_uv = padE(jnp.concatenate([u, v]), E_UV_PAD)
        ew_uv = padE(jnp.ones((2 * N_S,), jnp.float32), E_UV_PAD)
        u_g = padE(u, E_G_PAD)
        v_g = padE(v, E_G_PAD)

        def _padcols(x):
            c = x.shape[1]
            dp = 16 if c < 64 else 64
            return jnp.pad(x, ((0, 0), (0, dp - c))) if c != dp else x, dp

        bk_s = _make_bucketize(E_S_PAD, 10, 16384)(src_s, dst_s, ew_s)
        bk_t = _make_bucketize(E_T_PAD, 2, 6144)(src_t, dst_t, ew_t)
        bk_uv = _make_bucketize(E_UV_PAD, 2, 6144)(src_uv, dst_uv, ew_uv)

        def prop_s(x):
            xp, dp = _padcols(x)
            return _prop_sc(xp, bk_s, N_S, dp, 10, 16384)

        def prop_t(x):
            xp, dp = _padcols(x)
            return _prop_sc(xp, bk_t, N_T, dp, 2, 6144)

        def scatter_uv(xs):
            return _prop_sc(xs, bk_uv, N_T, FEAT, 2, 6144)

        deg64 = _prop_sc(jnp.zeros((8, FEAT), jnp.float32), bk_uv,
                         N_T, FEAT, 2, 6144, count_mode=True)

        def gather_uv(xt):
            return _gather_pair_sc(xt, u_g, v_g, N_S, FEAT)
    else:
        prop_s = lambda x: _prop_jnp(x, edge_index_s[0], edge_index_s[1],
                                     edge_weight_s)
        prop_t = lambda x: _prop_jnp(x, edge_index_t[0], edge_index_t[1],
                                     edge_weight_t)
        u = edge_index[0]
        v = edge_index[1]

        def scatter_uv(xs):
            agg = jnp.zeros((N_T, FEAT), jnp.float32)
            return agg.at[u].add(xs).at[v].add(xs)

        deg64 = scatter_uv(jnp.ones((N_S, FEAT), jnp.float32))

        def gather_uv(xt):
            return 0.5 * (xt[u] + xt[v])

    xs, _ = _cbr(x_s, prop_s, params['HL_EC'])
    xt, _ = _cbr(x_t, prop_t, params['HL_NC'])
    xs0 = xs
    xt0 = xt

    for i in range(4):
        last = i == 3
        for j, p in enumerate(params['NC'][i]):
            xt, cs_t = _cbr(xt, prop_t, p, x0=xt0,
                            colsum=(last and j == 1))
        for j, p in enumerate(params['EC'][i]):
            xs, cs_s = _cbr(xs, prop_s, p, x0=xs0,
                            colsum=(last and j == 1))
        if i < 3:
            agg = scatter_uv(xs)
            temp_xt = _scale_rows(agg, deg64)
            temp_xs = gather_uv(xt)
            xt_c = jnp.concatenate([xt, temp_xt], axis=-1)
            xs_c = jnp.concatenate([xs, temp_xs], axis=-1)
            xt, _ = _cbr(xt_c, prop_t, params['int_e2n'][i][0])
            xt, _ = _cbr(xt, prop_t, params['int_e2n'][i][1])
            xs, _ = _cbr(xs_c, prop_s, params['int_n2e'][i][0])
            xs, _ = _cbr(xs, prop_s, params['int_n2e'][i][1])
            xt0 = _lin(xt0, params['n0_proj'][i])
            xs0 = _lin(xs0, params['e0_proj'][i])

    w_out = params['out']['W']
    b_out = params['out']['b']
    w16 = jnp.pad(w_out, ((0, 0), (0, 16 - w_out.shape[1])))
    b16 = jnp.pad(b_out, (0, 16 - b_out.shape[0])).reshape(1, 16)
    out = _final(cs_s, cs_t, w16, b16)
    return out[0:1, 0:10]


# speedup vs baseline: 1.9355x; 1.2542x over previous
import functools

import jax
import jax.numpy as jnp
from jax import lax
from jax.experimental import pallas as pl
from jax.experimental.pallas import tpu as pltpu
from jax.experimental.pallas import tpu_sc as plsc

N_T = 10000
N_S = 160000
FEAT = 64
ALPHA = 0.5
ROWS = 2000

_USE_SC_PROP = True


def _dense_body(nx, stats, *refs):
    xs = refs[:nx]
    ws = refs[nx:2 * nx]
    b_ref = refs[2 * nx]
    y_ref = refs[2 * nx + 1]
    y = jnp.dot(xs[0][...], ws[0][...], preferred_element_type=jnp.float32)
    for i in range(1, nx):
        y = y + jnp.dot(xs[i][...], ws[i][...],
                        preferred_element_type=jnp.float32)
    y = y + b_ref[...]
    y_ref[...] = y
    if stats:
        st_ref = refs[2 * nx + 2]

        @pl.when(pl.program_id(0) == 0)
        def _():
            st_ref[...] = jnp.zeros_like(st_ref)

        s1 = jnp.sum(y, axis=0, keepdims=True)
        s2 = jnp.sum(y * y, axis=0, keepdims=True)
        st_ref[...] += jnp.concatenate([s1, s2], axis=0)


def _dense(x_list, w_list, b, stats):
    n = x_list[0].shape[0]
    nx = len(x_list)
    cout = w_list[0].shape[1]
    grid = n // ROWS
    in_specs = (
        [pl.BlockSpec((ROWS, x.shape[1]), lambda i: (i, 0)) for x in x_list]
        + [pl.BlockSpec(w.shape, lambda i: (0, 0)) for w in w_list]
        + [pl.BlockSpec((1, cout), lambda i: (0, 0))]
    )
    out_shape = [jax.ShapeDtypeStruct((n, cout), jnp.float32)]
    out_specs = [pl.BlockSpec((ROWS, cout), lambda i: (i, 0))]
    if stats:
        out_shape.append(jax.ShapeDtypeStruct((2, cout), jnp.float32))
        out_specs.append(pl.BlockSpec((2, cout), lambda i: (0, 0)))
    out = pl.pallas_call(
        functools.partial(_dense_body, nx, stats),
        grid=(grid,),
        in_specs=in_specs,
        out_specs=out_specs if stats else out_specs[0],
        out_shape=out_shape if stats else out_shape[0],
    )(*x_list, *w_list, b.reshape(1, cout))
    return out if stats else (out, None)


def _bn_body(n, mix, colsum, *refs):
    y_ref, st_ref, g_ref, bt_ref = refs[:4]
    idx = 4
    if mix:
        x0_ref = refs[idx]
        idx += 1
    o_ref = refs[idx]
    idx += 1
    st = st_ref[...]
    m = st[0:1, :] / n
    v = st[1:2, :] / n - m * m
    scale = g_ref[...] * lax.rsqrt(v + 1e-5)
    o = (y_ref[...] - m) * scale + bt_ref[...]
    o = jnp.maximum(o, 0.0)
    if mix:
        o = (1.0 - ALPHA) * o + ALPHA * x0_ref[...]
    o_ref[...] = o
    if colsum:
        cs_ref = refs[idx]

        @pl.when(pl.program_id(0) == 0)
        def _():
            cs_ref[...] = jnp.zeros_like(cs_ref)

        s1 = jnp.sum(o, axis=0, keepdims=True)
        cs_ref[...] += jnp.concatenate([s1, s1], axis=0)


def _bn_apply(y, st, g, bt, x0=None, colsum=False):
    n, c = y.shape
    grid = n // ROWS
    mix = x0 is not None
    in_specs = [
        pl.BlockSpec((ROWS, c), lambda i: (i, 0)),
        pl.BlockSpec((2, c), lambda i: (0, 0)),
        pl.BlockSpec((1, c), lambda i: (0, 0)),
        pl.BlockSpec((1, c), lambda i: (0, 0)),
    ]
    args = [y, st, g.reshape(1, c), bt.reshape(1, c)]
    if mix:
        in_specs.append(pl.BlockSpec((ROWS, c), lambda i: (i, 0)))
        args.append(x0)
    out_shape = [jax.ShapeDtypeStruct((n, c), jnp.float32)]
    out_specs = [pl.BlockSpec((ROWS, c), lambda i: (i, 0))]
    if colsum:
        out_shape.append(jax.ShapeDtypeStruct((2, c), jnp.float32))
        out_specs.append(pl.BlockSpec((2, c), lambda i: (0, 0)))
    out = pl.pallas_call(
        functools.partial(_bn_body, float(n), mix, colsum),
        grid=(grid,),
        in_specs=in_specs,
        out_specs=out_specs if colsum else out_specs[0],
        out_shape=out_shape if colsum else out_shape[0],
    )(*args)
    return out if colsum else (out, None)


def _scale_rows_body(a_ref, d_ref, o_ref):
    o_ref[...] = a_ref[...] / jnp.maximum(d_ref[...], 1.0)


def _scale_rows(a, d):
    n, c = a.shape
    return pl.pallas_call(
        _scale_rows_body,
        grid=(n // ROWS,),
        in_specs=[pl.BlockSpec((ROWS, c), lambda i: (i, 0)),
                  pl.BlockSpec((ROWS, c), lambda i: (i, 0))],
        out_specs=pl.BlockSpec((ROWS, c), lambda i: (i, 0)),
        out_shape=jax.ShapeDtypeStruct((n, c), jnp.float32),
    )(a, d)


def _final_body(ss_ref, st_ref, w_ref, b_ref, o_ref):
    pooled = jnp.concatenate(
        [ss_ref[0:1, :] / N_S, st_ref[0:1, :] / N_T], axis=1)
    pooled = jnp.broadcast_to(pooled, (8, 2 * FEAT))
    o_ref[...] = jnp.dot(pooled, w_ref[...],
                         preferred_element_type=jnp.float32) + b_ref[...]


def _final(sum_s, sum_t, w16, b16):
    return pl.pallas_call(
        _final_body,
        in_specs=[pl.BlockSpec((2, FEAT), lambda: (0, 0)),
                  pl.BlockSpec((2, FEAT), lambda: (0, 0)),
                  pl.BlockSpec((2 * FEAT, 16), lambda: (0, 0)),
                  pl.BlockSpec((1, 16), lambda: (0, 0))],
        out_specs=pl.BlockSpec((8, 16), lambda: (0, 0)),
        out_shape=jax.ShapeDtypeStruct((8, 16), jnp.float32),
    )(sum_s, sum_t, w16, b16)


def _prop_jnp(x, src, dst, ew):
    msg = ew[:, None] * x[src]
    return jnp.zeros_like(x).at[dst].add(msg)


_NC = 2
_NS = 16


def _take16(x, idx):
    return lax.gather(
        x, idx[:, None],
        lax.GatherDimensionNumbers(offset_dims=(),
                                   collapsed_slice_dims=(0,),
                                   start_index_map=(0,)),
        slice_sizes=(1,),
        mode=lax.GatherScatterMode.PROMISE_IN_BOUNDS)
_CHUNK = 2048
_BATCH = 128


@functools.lru_cache(maxsize=None)
def _make_bucketize(e_pad, nb, blk):
    nw = _NC * _NS
    tw = e_pad // nw
    c_sz = 2048 if tw % 2048 == 0 else 1024
    assert tw % c_sz == 0 and tw % 128 == 0
    nchunks = tw // c_sz
    nsg = c_sz // 64
    capr = tw + 128
    out_sz = nw * nb * capr + 512
    pcap = 192
    mesh = plsc.VectorSubcoreMesh(core_axis_name="c", subcore_axis_name="s",
                                  num_cores=_NC, num_subcores=_NS)

    @functools.partial(
        pl.kernel,
        out_type=(jax.ShapeDtypeStruct((out_sz,), jnp.int32),
                  jax.ShapeDtypeStruct((out_sz,), jnp.int32),
                  jax.ShapeDtypeStruct((out_sz,), jnp.float32),
                  jax.ShapeDtypeStruct((nw * 16,), jnp.int32)),
        mesh=mesh,
        compiler_params=pltpu.CompilerParams(use_tc_tiling_on_sc=False,
                                             needs_layout_passes=False),
        scratch_types=dict(
            srcb=pltpu.VMEM((c_sz,), jnp.int32),
            dstb=pltpu.VMEM((c_sz,), jnp.int32),
            ewb=pltpu.VMEM((c_sz,), jnp.float32),
            ps=pltpu.VMEM((nb, pcap), jnp.int32),
            pd=pltpu.VMEM((nb, pcap), jnp.int32),
            pw=pltpu.VMEM((nb, pcap), jnp.float32),
            cbuf=pltpu.VMEM((16,), jnp.int32),
        ),
    )
    def bk(src_hbm, dst_hbm, ew_hbm, bsrc, bdst, bew, cnts,
           srcb, dstb, ewb, ps, pd, pw, cbuf):
        cid = lax.axis_index("c")
        sid = lax.axis_index("s")
        w = sid * _NC + cid
        toff = w * tw

        def flush(b, nfl):
            base = pl.multiple_of((w * nb + b) * capr + nfl * 128, 128)
            pltpu.sync_copy(ps.at[b, pl.ds(0, 128)],
                            bsrc.at[pl.ds(base, 128)])
            pltpu.sync_copy(pd.at[b, pl.ds(0, 128)],
                            bdst.at[pl.ds(base, 128)])
            pltpu.sync_copy(pw.at[b, pl.ds(0, 128)],
                            bew.at[pl.ds(base, 128)])

        def shift(b):
            for g in range(4):
                o = pl.ds(128 + g * 16, 16)
                o0 = pl.ds(g * 16, 16)
                ps[b, o0] = ps[b, o]
                pd[b, o0] = pd[b, o]
                pw[b, o0] = pw[b, o]

        def chunk_body(c, carry):
            fills, nfls = carry
            eo = pl.multiple_of(toff + c * c_sz, 128)
            pltpu.sync_copy(src_hbm.at[pl.ds(eo, c_sz)], srcb)
            pltpu.sync_copy(dst_hbm.at[pl.ds(eo, c_sz)], dstb)
            pltpu.sync_copy(ew_hbm.at[pl.ds(eo, c_sz)], ewb)

            def sg_body(sg, carry):
                fills, nfls = carry
                last = jnp.full((16,), 15, jnp.int32)
                for gi in range(4):
                    o = pl.ds(sg * 64 + gi * 16, 16)
                    dv = dstb[o]
                    sv = srcb[o]
                    wv = ewb[o]
                    new_fills = []
                    for b in range(nb):
                        lo = b * blk
                        m = (dv >= lo) & (dv < lo + blk)
                        cum = plsc.cumsum(m.astype(jnp.int32))
                        pos = fills[b] + cum - 1
                        plsc.store_scatter(ps.at[b], [pos], sv, mask=m)
                        plsc.store_scatter(pd.at[b], [pos], dv - lo,
                                           mask=m)
                        plsc.store_scatter(pw.at[b], [pos], wv, mask=m)
                        tot = _take16(cum, last)
                        new_fills.append(fills[b] + tot)
                    fills = tuple(new_fills)
                new_fills, new_nfls = [], []
                for b in range(nb):
                    def do_flush(fn, b=b):
                        f, n = fn
                        flush(b, n)
                        shift(b)
                        return f - 128, n + 1

                    f, n = lax.cond(fills[b][0] >= 128, do_flush,
                                    lambda fn: fn, (fills[b], nfls[b]))
                    new_fills.append(f)
                    new_nfls.append(n)
                return tuple(new_fills), tuple(new_nfls)

            return lax.fori_loop(0, nsg, sg_body, (fills, nfls))

        zero_v = jnp.zeros((16,), jnp.int32)
        zero = jnp.int32(0)
        fills, nfls = lax.fori_loop(
            0, nchunks, chunk_body,
            (tuple(zero_v for _ in range(nb)),
             tuple(zero for _ in range(nb))))

        cvec = jnp.zeros((16,), jnp.int32)
        lane16 = lax.iota(jnp.int32, 16)
        for b in range(nb):
            def do_flush2(fn, b=b):
                f, n = fn
                flush(b, n)
                shift(b)
                return f - 128, n + 1

            f, n = lax.cond(fills[b][0] >= 128, do_flush2, lambda fn: fn,
                            (fills[b], nfls[b]))
            for g in range(8):
                o = pl.ds(g * 16, 16)
                keep = (lane16 + g * 16) < f
                ps[b, o] = jnp.where(keep, ps[b, o], 0)
                pd[b, o] = jnp.where(keep, pd[b, o], 0)
                pw[b, o] = jnp.where(keep, pw[b, o], jnp.float32(0.0))
            flush(b, n)
            cvec = jnp.where(lane16 == b, (n + 1) * 128, cvec)
        cbuf[pl.ds(0, 16)] = cvec
        pltpu.sync_copy(cbuf, cnts.at[pl.ds(w * 16, 16)])

    return bk


@functools.lru_cache(maxsize=None)
def _make_prop(n_rows, d, e_pad, nb, blk, count_mode, stage_x=False):
    nw = _NC * _NS
    tw = e_pad // nw
    capr = tw + 128
    rows_pt = blk // _NS
    assert rows_pt % _BATCH == 0 and nb % 2 == 0
    nfl = rows_pt // _BATCH
    out_pad = nb * blk
    nq = d // 16
    mesh = plsc.VectorSubcoreMesh(core_axis_name="c", subcore_axis_name="s",
                                  num_cores=_NC, num_subcores=_NS)

    @functools.partial(
        pl.kernel,
        out_type=jax.ShapeDtypeStruct((out_pad, d), jnp.float32),
        mesh=mesh,
        compiler_params=pltpu.CompilerParams(use_tc_tiling_on_sc=False,
                                             needs_layout_passes=False),
        scratch_types=dict(
            cntv=pltpu.VMEM((nw * 16,), jnp.int32),
            srcb2=pltpu.VMEM((4, _BATCH), jnp.int32),
            dstb2=pltpu.VMEM((4, _BATCH), jnp.int32),
            ewb2=pltpu.VMEM((4, _BATCH), jnp.float32),
            sdst=pltpu.VMEM((_BATCH,), jnp.int32),
            rows2=pltpu.VMEM((4, _BATCH, d), jnp.float32),
            zbuf=pltpu.VMEM((_BATCH, d), jnp.float32),
            accum=pltpu.MemorySpace.VMEM_SHARED((blk, d), jnp.float32),
            semr=pltpu.SemaphoreType.DMA((4,)),
            semg=pltpu.SemaphoreType.DMA((4,)),
            **(dict(xsh=pltpu.MemorySpace.VMEM_SHARED((n_rows, d),
                                                      jnp.float32))
               if stage_x else {}),
        ),
    )
    def prop_k(x_hbm, bsrc, bdst, bew, cnts, out_hbm,
               cntv, srcb2, dstb2, ewb2, sdst, rows2, zbuf, accum,
               semr, semg, xsh=None):
        cid = lax.axis_index("c")
        sid = lax.axis_index("s")
        pltpu.sync_copy(cnts, cntv)
        if stage_x:
            nst = n_rows // _NS
            pltpu.sync_copy(x_hbm.at[pl.ds(sid * nst, nst), :],
                            xsh.at[pl.ds(sid * nst, nst), :])
        xsrc = xsh if stage_x else x_hbm
        lane16 = lax.iota(jnp.int32, 16)

        def zb_init(r, carry):
            for q in range(nq):
                zbuf[r, pl.ds(q * 16, 16)] = jnp.zeros((16,), jnp.float32)
            return carry

        lax.fori_loop(0, _BATCH, zb_init, 0)

        def rec_issue(rbase, j, p):
            base = pl.multiple_of(rbase + j * _BATCH, _BATCH)
            pltpu.async_copy(bsrc.at[pl.ds(base, _BATCH)], srcb2.at[p],
                             semr.at[p])
            pltpu.async_copy(bdst.at[pl.ds(base, _BATCH)], dstb2.at[p],
                             semr.at[p])
            pltpu.async_copy(bew.at[pl.ds(base, _BATCH)], ewb2.at[p],
                             semr.at[p])

        def rec_wait(rbase, j, p):
            base = pl.multiple_of(rbase + j * _BATCH, _BATCH)
            pltpu.make_async_copy(bsrc.at[pl.ds(base, _BATCH)],
                                  srcb2.at[p], semr.at[p]).wait()
            pltpu.make_async_copy(bdst.at[pl.ds(base, _BATCH)],
                                  dstb2.at[p], semr.at[p]).wait()
            pltpu.make_async_copy(bew.at[pl.ds(base, _BATCH)],
                                  ewb2.at[p], semr.at[p]).wait()

        def g_issue(p):
            if not count_mode:
                pltpu.async_copy(xsrc.at[srcb2.at[p]], rows2.at[p],
                                 semg.at[p])

        def g_wait(p):
            if not count_mode:
                pltpu.make_async_copy(xsrc.at[srcb2.at[p]], rows2.at[p],
                                      semg.at[p]).wait()

        def finish(p):
            for g in range(_BATCH // 16):
                o = pl.ds(g * 16, 16)
                sdst[o] = dstb2[p, o]

            def scale(g, carry):
                wv = ewb2[p, pl.ds(g * 16, 16)]
                for i in range(16):
                    wsp = _take16(wv, jnp.full((16,), i, jnp.int32))
                    r = g * 16 + i
                    for q in range(nq):
                        o = pl.ds(q * 16, 16)
                        if count_mode:
                            rows2[p, r, o] = wsp
                        else:
                            rows2[p, r, o] = rows2[p, r, o] * wsp
                return carry

            lax.fori_loop(0, _BATCH // 16, scale, 0)
            pltpu.sync_copy(rows2.at[p], accum.at[sdst], add=True)

        def region(rbase, n):
            for k in range(4):
                rec_issue(rbase, k, k)
            for k in range(3):
                def pro(_, k=k):
                    rec_wait(rbase, k, k)
                    g_issue(k)
                    return 0

                lax.cond(k < n, pro, lambda _: 0, 0)

            def it(j, c):
                def start_next(_):
                    rec_wait(rbase, j + 3, (j + 3) % 4)
                    g_issue((j + 3) % 4)
                    return 0

                lax.cond(j + 3 < n, start_next, lambda _: 0, 0)
                p = j % 4
                g_wait(p)
                finish(p)
                rec_issue(rbase, j + 4, p)
                return c

            lax.fori_loop(0, n, it, 0)
            for k in range(4):
                rec_wait(rbase, n + k, (n + k) % 4)

        def block_body(bi, carry):
            b = bi * 2 + cid
            lo = b * blk

            def zero(j, c):
                r0 = sid * rows_pt + j * _BATCH
                pltpu.sync_copy(zbuf, accum.at[pl.ds(r0, _BATCH), :])
                return c

            lax.fori_loop(0, nfl, zero, 0)
            plsc.subcore_barrier()

            for wi in range(2):
                w = 2 * sid + wi
                cv = cntv[pl.ds(w * 16, 16)]
                n = jnp.max(jnp.where(lane16 == b, cv, 0)) // _BATCH
                region((w * nb + b) * capr, n)
            plsc.subcore_barrier()

            def fl(j, c):
                r0 = sid * rows_pt + j * _BATCH
                pltpu.sync_copy(accum.at[pl.ds(r0, _BATCH), :],
                                out_hbm.at[pl.ds(lo + r0, _BATCH), :])
                return c

            lax.fori_loop(0, nfl, fl, 0)
            plsc.subcore_barrier()
            return carry

        lax.fori_loop(0, nb // 2, block_body, 0)

    return prop_k


def _prop_sc(x, buckets, n_rows, d, nb, blk, count_mode=False,
             stage_x=False):
    bsrc, bdst, bew, cnts = buckets
    nw = _NC * _NS
    capr = (bsrc.shape[0] - 512) // (nw * nb)
    e_pad = (capr - 128) * nw
    k = _make_prop(n_rows, d, e_pad, nb, blk, count_mode, stage_x)
    out = k(x, bsrc, bdst, bew, cnts)
    return out[:n_rows]


@functools.lru_cache(maxsize=None)
def _make_gather_pair(n_out, n_tab, d, e_pad):
    nw = _NC * _NS
    per_w = e_pad // nw
    assert per_w % _BATCH == 0
    nbat = per_w // _BATCH
    nflat = (_BATCH * d) // 16
    mesh = plsc.VectorSubcoreMesh(core_axis_name="c", subcore_axis_name="s",
                                  num_cores=_NC, num_subcores=_NS)

    @functools.partial(
        pl.kernel,
        out_type=jax.ShapeDtypeStruct((e_pad, d), jnp.float32),
        mesh=mesh,
        compiler_params=pltpu.CompilerParams(use_tc_tiling_on_sc=False,
                                             needs_layout_passes=False),
        scratch_types=dict(
            ub=pltpu.VMEM((_BATCH,), jnp.int32),
            vb=pltpu.VMEM((_BATCH,), jnp.int32),
            rowsa=pltpu.VMEM((_BATCH, d), jnp.float32),
            rowsb=pltpu.VMEM((_BATCH, d), jnp.float32),
            xsh=pltpu.MemorySpace.VMEM_SHARED((n_tab, d), jnp.float32),
            sem=pltpu.SemaphoreType.DMA,
            sem2=pltpu.SemaphoreType.DMA,
        ),
    )
    def gather_k(xt_hbm, u_hbm, v_hbm, out_hbm, ub, vb, rowsa, rowsb,
                 xsh, sem, sem2):
        cid = lax.axis_index("c")
        sid = lax.axis_index("s")
        wid = sid * _NC + cid
        base = wid * per_w
        nst = n_tab // _NS
        pltpu.sync_copy(xt_hbm.at[pl.ds(sid * nst, nst), :],
                        xsh.at[pl.ds(sid * nst, nst), :])
        plsc.subcore_barrier()

        def bat(j, carry):
            eo = pl.multiple_of(base + j * _BATCH, _BATCH)
            pltpu.sync_copy(u_hbm.at[pl.ds(eo, _BATCH)], ub)
            pltpu.sync_copy(v_hbm.at[pl.ds(eo, _BATCH)], vb)
            cpa = pltpu.async_copy(xsh.at[ub], rowsa, sem)
            cpb = pltpu.async_copy(xsh.at[vb], rowsb, sem2)
            cpa.wait()
            cpb.wait()

            def addf(g, c):
                r = g // nq_
                o = pl.ds((g % nq_) * 16, 16)
                rowsa[r, o] = 0.5 * (rowsa[r, o] + rowsb[r, o])
                return c

            nq_ = d // 16
            lax.fori_loop(0, nflat, addf, 0)
            pltpu.sync_copy(rowsa, out_hbm.at[pl.ds(eo, _BATCH), :])
            return carry

        lax.fori_loop(0, nbat, bat, 0)

    return gather_k


def _gather_pair_sc(xt, u_p, v_p, n_out, d):
    e_pad = u_p.shape[0]
    k = _make_gather_pair(n_out, xt.shape[0], d, e_pad)
    return k(xt, u_p, v_p)[:n_out]


def _cbr(x, prop_fn, p, x0=None, colsum=False):
    ws = p['W']
    if len(ws) > 1:
        pr = prop_fn(x)
        w1 = ws[1]
        if pr.shape[1] != w1.shape[0]:
            w1p = jnp.pad(w1, ((0, pr.shape[1] - w1.shape[0]), (0, 0)))
        else:
            w1p = w1
        y, st = _dense([x, pr], [ws[0] + ws[1], -w1p], p['b'], True)
    else:
        y, st = _dense([x], [ws[0]], p['b'], True)
    return _bn_apply(y, st, p['g'], p['bt'], x0=x0, colsum=colsum)


def _lin(x, p):
    y, _ = _dense([x], [p['W'][0]], p['b'], False)
    return y


def kernel(x_s, x_t, edge_index_s, edge_weight_s, edge_index_t,
           edge_weight_t, edge_index, params):
    if _USE_SC_PROP:
        def padE(a, epad, dtype=None):
            return jnp.pad(a, (0, epad - a.shape[0]))

        E_S_PAD, E_T_PAD, E_UV_PAD = 524288, 360448, 327680
        E_G_PAD = 163840
        src_s = padE(edge_index_s[0], E_S_PAD)
        dst_s = padE(edge_index_s[1], E_S_PAD)
        ew_s = padE(edge_weight_s, E_S_PAD)
        src_t = padE(edge_index_t[0], E_T_PAD)
        dst_t = padE(edge_index_t[1], E_T_PAD)
        ew_t = padE(edge_weight_t, E_T_PAD)
        u = edge_index[0]
        v = edge_index[1]
        ar = jnp.arange(N_S, dtype=jnp.int32)
        src_uv = padE(jnp.concatenate([ar, ar]), E_UV_PAD)
        dst_uv = padE(jnp.concatenate([u, v]), E_UV_PAD)
        ew_uv = padE(jnp.ones((2 * N_S,), jnp.float32), E_UV_PAD)
        u_g = padE(u, E_G_PAD)
        v_g = padE(v, E_G_PAD)

        def _padcols(x):
            c = x.shape[1]
            dp = 16 if c < 64 else 64
            return jnp.pad(x, ((0, 0), (0, dp - c))) if c != dp else x, dp

        bk_s = _make_bucketize(E_S_PAD, 10, 16384)(src_s, dst_s, ew_s)
        bk_t = _make_bucketize(E_T_PAD, 2, 6144)(src_t, dst_t, ew_t)
        bk_uv = _make_bucketize(E_UV_PAD, 2, 6144)(src_uv, dst_uv, ew_uv)

        def prop_s(x):
            xp, dp = _padcols(x)
            return _prop_sc(xp, bk_s, N_S, dp, 10, 16384)

        def prop_t(x):
            xp, dp = _padcols(x)
            return _prop_sc(xp, bk_t, N_T, dp, 2, 6144, stage_x=True)

        def scatter_uv(xs):
            return _prop_sc(xs, bk_uv, N_T, FEAT, 2, 6144)

        deg64 = _prop_sc(jnp.zeros((8, FEAT), jnp.float32), bk_uv,
                         N_T, FEAT, 2, 6144, count_mode=True)

        def gather_uv(xt):
            return _gather_pair_sc(xt, u_g, v_g, N_S, FEAT)
    else:
        prop_s = lambda x: _prop_jnp(x, edge_index_s[0], edge_index_s[1],
                                     edge_weight_s)
        prop_t = lambda x: _prop_jnp(x, edge_index_t[0], edge_index_t[1],
                                     edge_weight_t)
        u = edge_index[0]
        v = edge_index[1]

        def scatter_uv(xs):
            agg = jnp.zeros((N_T, FEAT), jnp.float32)
            return agg.at[u].add(xs).at[v].add(xs)

        deg64 = scatter_uv(jnp.ones((N_S, FEAT), jnp.float32))

        def gather_uv(xt):
            return 0.5 * (xt[u] + xt[v])

    xs, _ = _cbr(x_s, prop_s, params['HL_EC'])
    xt, _ = _cbr(x_t, prop_t, params['HL_NC'])
    xs0 = xs
    xt0 = xt

    for i in range(4):
        last = i == 3
        for j, p in enumerate(params['NC'][i]):
            xt, cs_t = _cbr(xt, prop_t, p, x0=xt0,
                            colsum=(last and j == 1))
        for j, p in enumerate(params['EC'][i]):
            xs, cs_s = _cbr(xs, prop_s, p, x0=xs0,
                            colsum=(last and j == 1))
        if i < 3:
            agg = scatter_uv(xs)
            temp_xt = _scale_rows(agg, deg64)
            temp_xs = gather_uv(xt)
            xt_c = jnp.concatenate([xt, temp_xt], axis=-1)
            xs_c = jnp.concatenate([xs, temp_xs], axis=-1)
            xt, _ = _cbr(xt_c, prop_t, params['int_e2n'][i][0])
            xt, _ = _cbr(xt, prop_t, params['int_e2n'][i][1])
            xs, _ = _cbr(xs_c, prop_s, params['int_n2e'][i][0])
            xs, _ = _cbr(xs, prop_s, params['int_n2e'][i][1])
            xt0 = _lin(xt0, params['n0_proj'][i])
            xs0 = _lin(xs0, params['e0_proj'][i])

    w_out = params['out']['W']
    b_out = params['out']['b']
    w16 = jnp.pad(w_out, ((0, 0), (0, 16 - w_out.shape[1])))
    b16 = jnp.pad(b_out, (0, 16 - b_out.shape[0])).reshape(1, 16)
    out = _final(cs_s, cs_t, w16, b16)
    return out[0:1, 0:10]
